# Initial kernel scaffold; baseline (speedup 1.0000x reference)
import jax
import jax.numpy as jnp
from jax.experimental import pallas as pl


def kernel(x, edge_index, W1, b1, g1, be1, W2, b2, g2, be2):
    raise NotImplementedError("write your pallas kernel here")


# speedup vs baseline: 4.2019x; 4.2019x over previous
import functools

import jax
import jax.numpy as jnp
from jax import lax
from jax.experimental import pallas as pl
from jax.experimental.pallas import tpu as pltpu
from jax.experimental.pallas import tpu_sc as plsc

N = 10000
D = 256
E = 160000

NC = 2
NS = 16
NW = NC * NS
RNG = 312
RNGL = N - (NW - 1) * RNG
ACCR = 336
TS = ACCR - 1
SCH = 2000
NSCH = E // SCH
STG = 4096
FLUSH = 2048
EPW = E + FLUSH
GCH = 64

RB = 1000

_mesh = plsc.VectorSubcoreMesh(
    core_axis_name="c", subcore_axis_name="s", num_cores=NC, num_subcores=NS
)
_params = pltpu.CompilerParams(needs_layout_passes=False)


def _worker(c, t):
    return c * NS + t


@functools.partial(
    pl.kernel,
    out_type=[
        jax.ShapeDtypeStruct((NW * EPW,), jnp.int32),
        jax.ShapeDtypeStruct((NW * EPW,), jnp.int32),
        jax.ShapeDtypeStruct((NW * 16,), jnp.int32),
        jax.ShapeDtypeStruct((N,), jnp.float32),
    ],
    mesh=_mesh,
    compiler_params=_params,
    scratch_types=[
        pltpu.VMEM((SCH,), jnp.int32),
        pltpu.VMEM((SCH,), jnp.int32),
        pltpu.VMEM((STG,), jnp.int32),
        pltpu.VMEM((STG,), jnp.int32),
        pltpu.VMEM((ACCR,), jnp.float32),
    ],
)
def _compact_kernel(src_hbm, dst_hbm, csrc_hbm, cdst_hbm, cnts_hbm, deg_hbm,
                    sbuf, dbuf, stg_s, stg_d, degloc):
    c = lax.axis_index("c")
    t = lax.axis_index("s")
    w = _worker(c, t)
    lo = w * RNG
    hi = jnp.where(w == NW - 1, N, lo + RNG)
    base = w * EPW
    ones16 = jnp.ones((16,), jnp.float32)
    zero16 = jnp.zeros((16,), jnp.float32)

    def zdeg(i, _):
        degloc[pl.ds(i * 16, 16)] = zero16
        return 0

    lax.fori_loop(0, ACCR // 16, zdeg, 0)

    def chunk(k, carry):
        off0, hoff0 = carry
        pltpu.sync_copy(src_hbm.at[pl.ds(k * SCH, SCH)], sbuf)
        pltpu.sync_copy(dst_hbm.at[pl.ds(k * SCH, SCH)], dbuf)

        def grp(j, off):
            d16 = dbuf[pl.ds(j * 16, 16)]
            s16 = sbuf[pl.ds(j * 16, 16)]
            dl = d16 - lo
            m = (d16 >= lo) & (d16 < hi)
            plsc.store_compressed(stg_s.at[pl.ds(off, 16)], s16, mask=m)
            plsc.store_compressed(stg_d.at[pl.ds(off, 16)], dl, mask=m)
            plsc.addupdate_scatter(
                degloc, [jnp.where(m, dl, TS)], ones16, mask=m)
            pc = plsc.all_reduce_population_count(m)
            return off + pc[0]

        off = lax.fori_loop(0, SCH // 16, grp, off0)

        def do_flush(a):
            o, h = a
            ho = pl.multiple_of(base + h, 8)
            pltpu.sync_copy(stg_s.at[pl.ds(0, FLUSH)],
                            csrc_hbm.at[pl.ds(ho, FLUSH)])
            pltpu.sync_copy(stg_d.at[pl.ds(0, FLUSH)],
                            cdst_hbm.at[pl.ds(ho, FLUSH)])

            def shift(i, _):
                stg_s[pl.ds(i * 16, 16)] = stg_s[pl.ds(FLUSH + i * 16, 16)]
                stg_d[pl.ds(i * 16, 16)] = stg_d[pl.ds(FLUSH + i * 16, 16)]
                return 0

            lax.fori_loop(0, FLUSH // 16, shift, 0)
            return (o - FLUSH, h + FLUSH)

        return lax.cond(off >= FLUSH, do_flush, lambda a: a, (off, hoff0))

    off, hoff = lax.fori_loop(0, NSCH, chunk, (jnp.int32(0), jnp.int32(0)))

    hof = pl.multiple_of(base + hoff, 8)
    pltpu.sync_copy(stg_s.at[pl.ds(0, FLUSH)],
                    csrc_hbm.at[pl.ds(hof, FLUSH)])
    pltpu.sync_copy(stg_d.at[pl.ds(0, FLUSH)],
                    cdst_hbm.at[pl.ds(hof, FLUSH)])

    sbuf[pl.ds(0, 16)] = jnp.broadcast_to(hoff + off, (16,))
    pltpu.sync_copy(sbuf.at[pl.ds(0, 16)], cnts_hbm.at[pl.ds(w * 16, 16)])

    @pl.when(w < NW - 1)
    def _():
        pltpu.sync_copy(degloc.at[pl.ds(0, RNG)],
                        deg_hbm.at[pl.ds(lo, RNG)])

    @pl.when(w == NW - 1)
    def _():
        pltpu.sync_copy(degloc.at[pl.ds(0, RNGL)],
                        deg_hbm.at[pl.ds(lo, RNGL)])


@functools.partial(
    pl.kernel,
    out_type=jax.ShapeDtypeStruct((N, D), jnp.float32),
    mesh=_mesh,
    compiler_params=_params,
    scratch_types=[
        pltpu.VMEM((160,), jnp.int32),
        pltpu.VMEM((160,), jnp.int32),
        pltpu.VMEM((2, GCH, D), jnp.float32),
        pltpu.VMEM((ACCR, D), jnp.float32),
        pltpu.SemaphoreType.DMA,
        pltpu.SemaphoreType.DMA,
    ],
)
def _agg_kernel(z_hbm, csrc_hbm, cdst_hbm, cnts_hbm, agg_hbm,
                srcb, dstb, rows, acc, sem_a, sem_b):
    c = lax.axis_index("c")
    t = lax.axis_index("s")
    w = _worker(c, t)
    base = w * EPW
    iota16 = lax.iota(jnp.int32, 16)
    zero16 = jnp.zeros((16,), jnp.float32)
    lanes = D // 16

    pltpu.sync_copy(cnts_hbm.at[pl.ds(w * 16, 16)], srcb.at[pl.ds(0, 16)])
    cnt = srcb[pl.ds(0, 16)][0]
    nch = (cnt + (GCH - 1)) // GCH

    def zacc(i, _):
        r = i // lanes
        j = i - r * lanes
        acc[r, pl.ds(j * 16, 16)] = zero16
        return 0

    lax.fori_loop(0, ACCR * lanes, zacc, 0)

    def load_idx(g, hb):
        go = pl.multiple_of(base + g * GCH, 8)
        pltpu.sync_copy(csrc_hbm.at[pl.ds(go, GCH)],
                        srcb.at[pl.ds(GCH * hb, GCH)])
        pltpu.sync_copy(cdst_hbm.at[pl.ds(go, GCH)],
                        dstb.at[pl.ds(GCH * hb, GCH)])
        for q in range(GCH // 16):
            sl = pl.ds(GCH * hb + q * 16, 16)
            pos = g * GCH + q * 16 + iota16
            mm = pos < cnt
            srcb[sl] = jnp.where(mm, srcb[sl], iota16 & 7)
            dstb[sl] = jnp.where(mm, dstb[sl], TS)

    def start_gather(hb, sem):
        pltpu.async_copy(z_hbm.at[srcb.at[pl.ds(GCH * hb, GCH)]],
                         rows.at[hb], sem)

    def wait_gather(hb, sem):
        pltpu.make_async_copy(z_hbm.at[srcb.at[pl.ds(GCH * hb, GCH)]],
                              rows.at[hb], sem).wait()

    @pl.when(nch > 0)
    def _():
        load_idx(0, 0)
        start_gather(0, sem_a)

    @pl.when(nch > 1)
    def _():
        load_idx(1, 1)
        start_gather(1, sem_b)

    def pair(p, _):
        for hb in range(2):
            g = 2 * p + hb
            sem = sem_a if hb == 0 else sem_b

            @pl.when(g < nch)
            def _():
                wait_gather(hb, sem)

                def acc_e(e, _):
                    sl = dstb[pl.ds(GCH * hb + e, 16)][0]
                    for j in range(lanes):
                        acc[sl, pl.ds(j * 16, 16)] = (
                            acc[sl, pl.ds(j * 16, 16)]
                            + rows[hb, e, pl.ds(j * 16, 16)])
                    return 0

                lax.fori_loop(0, GCH, acc_e, 0)

                @pl.when(g + 2 < nch)
                def _():
                    load_idx(g + 2, hb)
                    start_gather(hb, sem)

        return 0

    lax.fori_loop(0, (nch + 1) // 2, pair, 0)

    @pl.when(w < NW - 1)
    def _():
        pltpu.sync_copy(acc.at[pl.ds(0, RNG)],
                        agg_hbm.at[pl.ds(w * RNG, RNG)])

    @pl.when(w == NW - 1)
    def _():
        pltpu.sync_copy(acc.at[pl.ds(0, RNGL)],
                        agg_hbm.at[pl.ds(w * RNG, RNGL)])


def _mm_body(deg_ref, x_ref, w_ref, z_ref):
    dv = lax.rsqrt(1.0 + deg_ref[...])
    z_ref[...] = jnp.dot(
        x_ref[...] * dv, w_ref[...], preferred_element_type=jnp.float32
    )


def _mm(deg, x, w):
    return pl.pallas_call(
        _mm_body,
        grid=(N // RB,),
        in_specs=[
            pl.BlockSpec((RB, 1), lambda i: (i, 0)),
            pl.BlockSpec((RB, D), lambda i: (i, 0)),
            pl.BlockSpec((D, D), lambda i: (0, 0)),
        ],
        out_specs=pl.BlockSpec((RB, D), lambda i: (i, 0)),
        out_shape=jax.ShapeDtypeStruct((N, D), jnp.float32),
    )(deg, x, w)


def _ln_relu(pre, g_ref, be_ref):
    mu = jnp.mean(pre, axis=-1, keepdims=True)
    xc = pre - mu
    var = jnp.mean(xc * xc, axis=-1, keepdims=True)
    y = xc * lax.rsqrt(var + 1e-5) * g_ref[...] + be_ref[...]
    return jnp.maximum(y, 0.0)


def _ep1_body(deg_ref, agg_ref, z_ref, b_ref, g_ref, be_ref,
              w2_ref, h1_ref, z2_ref):
    dv = lax.rsqrt(1.0 + deg_ref[...])
    pre = dv * (agg_ref[...] + z_ref[...]) + b_ref[...]
    h1 = _ln_relu(pre, g_ref, be_ref)
    h1_ref[...] = h1
    z2_ref[...] = jnp.dot(
        h1 * dv, w2_ref[...], preferred_element_type=jnp.float32
    )


def _ep1(deg, agg1, z1, b1, g1, be1, W2):
    return pl.pallas_call(
        _ep1_body,
        grid=(N // RB,),
        in_specs=[
            pl.BlockSpec((RB, 1), lambda i: (i, 0)),
            pl.BlockSpec((RB, D), lambda i: (i, 0)),
            pl.BlockSpec((RB, D), lambda i: (i, 0)),
            pl.BlockSpec((1, D), lambda i: (0, 0)),
            pl.BlockSpec((1, D), lambda i: (0, 0)),
            pl.BlockSpec((1, D), lambda i: (0, 0)),
            pl.BlockSpec((D, D), lambda i: (0, 0)),
        ],
        out_specs=[
            pl.BlockSpec((RB, D), lambda i: (i, 0)),
            pl.BlockSpec((RB, D), lambda i: (i, 0)),
        ],
        out_shape=[jax.ShapeDtypeStruct((N, D), jnp.float32)] * 2,
    )(deg, agg1, z1, b1, g1, be1, W2)


def _ep2_body(deg_ref, agg_ref, z_ref, b_ref, g_ref, be_ref,
              h1_ref, out_ref):
    dv = lax.rsqrt(1.0 + deg_ref[...])
    pre = dv * (agg_ref[...] + z_ref[...]) + b_ref[...]
    h2 = _ln_relu(pre, g_ref, be_ref)
    out_ref[...] = h1_ref[...] + h2


def _ep2(deg, agg2, z2, b2, g2, be2, h1):
    return pl.pallas_call(
        _ep2_body,
        grid=(N // RB,),
        in_specs=[
            pl.BlockSpec((RB, 1), lambda i: (i, 0)),
            pl.BlockSpec((RB, D), lambda i: (i, 0)),
            pl.BlockSpec((RB, D), lambda i: (i, 0)),
            pl.BlockSpec((1, D), lambda i: (0, 0)),
            pl.BlockSpec((1, D), lambda i: (0, 0)),
            pl.BlockSpec((1, D), lambda i: (0, 0)),
            pl.BlockSpec((RB, D), lambda i: (i, 0)),
        ],
        out_specs=pl.BlockSpec((RB, D), lambda i: (i, 0)),
        out_shape=jax.ShapeDtypeStruct((N, D), jnp.float32),
    )(deg, agg2, z2, b2, g2, be2, h1)


def kernel(x, edge_index, W1, b1, g1, be1, W2, b2, g2, be2):
    src_flat = edge_index[0]
    dst_flat = edge_index[1]

    csrc, cdst, cnts, deg = _compact_kernel(src_flat, dst_flat)
    deg2 = deg.reshape(N, 1)

    z1 = _mm(deg2, x, W1)
    agg1 = _agg_kernel(z1, csrc, cdst, cnts)
    h1, z2 = _ep1(deg2, agg1, z1, b1[None], g1[None], be1[None], W2)
    agg2 = _agg_kernel(z2, csrc, cdst, cnts)
    return _ep2(deg2, agg2, z2, b2[None], g2[None], be2[None], h1)


# speedup vs baseline: 4.8151x; 1.1459x over previous
import functools

import jax
import jax.numpy as jnp
from jax import lax
from jax.experimental import pallas as pl
from jax.experimental.pallas import tpu as pltpu
from jax.experimental.pallas import tpu_sc as plsc

N = 10000
D = 256
E = 160000

NC = 2
NS = 16
NW = NC * NS
RNG = 312
RNGL = N - (NW - 1) * RNG
ACCR = 336
TS = ACCR - 1
SCH = 2000
NSCH = E // SCH
STG = 4096
FLUSH = 2048
EPW = E + FLUSH
GCH = 64

RB = 1000

_mesh = plsc.VectorSubcoreMesh(
    core_axis_name="c", subcore_axis_name="s", num_cores=NC, num_subcores=NS
)
_params = pltpu.CompilerParams(needs_layout_passes=False)


def _worker(c, t):
    return c * NS + t


@functools.partial(
    pl.kernel,
    out_type=[
        jax.ShapeDtypeStruct((NW * EPW,), jnp.int32),
        jax.ShapeDtypeStruct((NW * EPW,), jnp.int32),
        jax.ShapeDtypeStruct((NW * 16,), jnp.int32),
        jax.ShapeDtypeStruct((N,), jnp.float32),
    ],
    mesh=_mesh,
    compiler_params=_params,
    scratch_types=[
        pltpu.VMEM((SCH,), jnp.int32),
        pltpu.VMEM((SCH,), jnp.int32),
        pltpu.VMEM((STG,), jnp.int32),
        pltpu.VMEM((STG,), jnp.int32),
        pltpu.VMEM((ACCR,), jnp.float32),
    ],
)
def _compact_kernel(src_hbm, dst_hbm, csrc_hbm, cdst_hbm, cnts_hbm, deg_hbm,
                    sbuf, dbuf, stg_s, stg_d, degloc):
    c = lax.axis_index("c")
    t = lax.axis_index("s")
    w = _worker(c, t)
    lo = w * RNG
    hi = jnp.where(w == NW - 1, N, lo + RNG)
    base = w * EPW
    ones16 = jnp.ones((16,), jnp.float32)
    zero16 = jnp.zeros((16,), jnp.float32)

    def zdeg(i, _):
        degloc[pl.ds(i * 16, 16)] = zero16
        return 0

    lax.fori_loop(0, ACCR // 16, zdeg, 0)

    def chunk(k, carry):
        off0, hoff0 = carry
        pltpu.sync_copy(src_hbm.at[pl.ds(k * SCH, SCH)], sbuf)
        pltpu.sync_copy(dst_hbm.at[pl.ds(k * SCH, SCH)], dbuf)

        def grp(j, off):
            d16 = dbuf[pl.ds(j * 16, 16)]
            s16 = sbuf[pl.ds(j * 16, 16)]
            dl = d16 - lo
            m = (d16 >= lo) & (d16 < hi)
            plsc.store_compressed(stg_s.at[pl.ds(off, 16)], s16, mask=m)
            plsc.store_compressed(stg_d.at[pl.ds(off, 16)], dl, mask=m)
            plsc.addupdate_scatter(
                degloc, [jnp.where(m, dl, TS)], ones16, mask=m)
            pc = plsc.all_reduce_population_count(m)
            return off + pc[0]

        off = lax.fori_loop(0, SCH // 16, grp, off0)

        def do_flush(a):
            o, h = a
            ho = pl.multiple_of(base + h, 8)
            pltpu.sync_copy(stg_s.at[pl.ds(0, FLUSH)],
                            csrc_hbm.at[pl.ds(ho, FLUSH)])
            pltpu.sync_copy(stg_d.at[pl.ds(0, FLUSH)],
                            cdst_hbm.at[pl.ds(ho, FLUSH)])

            def shift(i, _):
                stg_s[pl.ds(i * 16, 16)] = stg_s[pl.ds(FLUSH + i * 16, 16)]
                stg_d[pl.ds(i * 16, 16)] = stg_d[pl.ds(FLUSH + i * 16, 16)]
                return 0

            lax.fori_loop(0, FLUSH // 16, shift, 0)
            return (o - FLUSH, h + FLUSH)

        return lax.cond(off >= FLUSH, do_flush, lambda a: a, (off, hoff0))

    off, hoff = lax.fori_loop(0, NSCH, chunk, (jnp.int32(0), jnp.int32(0)))

    hof = pl.multiple_of(base + hoff, 8)
    pltpu.sync_copy(stg_s.at[pl.ds(0, FLUSH)],
                    csrc_hbm.at[pl.ds(hof, FLUSH)])
    pltpu.sync_copy(stg_d.at[pl.ds(0, FLUSH)],
                    cdst_hbm.at[pl.ds(hof, FLUSH)])

    sbuf[pl.ds(0, 16)] = jnp.broadcast_to(hoff + off, (16,))
    pltpu.sync_copy(sbuf.at[pl.ds(0, 16)], cnts_hbm.at[pl.ds(w * 16, 16)])

    @pl.when(w < NW - 1)
    def _():
        pltpu.sync_copy(degloc.at[pl.ds(0, RNG)],
                        deg_hbm.at[pl.ds(lo, RNG)])

    @pl.when(w == NW - 1)
    def _():
        pltpu.sync_copy(degloc.at[pl.ds(0, RNGL)],
                        deg_hbm.at[pl.ds(lo, RNGL)])


@functools.partial(
    pl.kernel,
    out_type=jax.ShapeDtypeStruct((N * D,), jnp.float32),
    mesh=_mesh,
    compiler_params=_params,
    scratch_types=[
        pltpu.VMEM((160,), jnp.int32),
        pltpu.VMEM((160,), jnp.int32),
        pltpu.VMEM((2, GCH, D), jnp.float32),
        pltpu.VMEM((ACCR * D,), jnp.float32),
        pltpu.SemaphoreType.DMA,
        pltpu.SemaphoreType.DMA,
    ],
)
def _agg_kernel(z_hbm, csrc_hbm, cdst_hbm, cnts_hbm, agg_hbm,
                srcb, dstb, rows, acc, sem_a, sem_b):
    c = lax.axis_index("c")
    t = lax.axis_index("s")
    w = _worker(c, t)
    base = w * EPW
    iota16 = lax.iota(jnp.int32, 16)
    zero16 = jnp.zeros((16,), jnp.float32)
    lanes = D // 16

    pltpu.sync_copy(cnts_hbm.at[pl.ds(w * 16, 16)], srcb.at[pl.ds(0, 16)])
    cnt = srcb[pl.ds(0, 16)][0]
    nch = (cnt + (GCH - 1)) // GCH

    def zacc(i, _):
        acc[pl.ds(i * 16, 16)] = zero16
        return 0

    lax.fori_loop(0, ACCR * lanes, zacc, 0)

    def load_idx(g, hb):
        go = pl.multiple_of(base + g * GCH, 8)
        pltpu.sync_copy(csrc_hbm.at[pl.ds(go, GCH)],
                        srcb.at[pl.ds(GCH * hb, GCH)])
        pltpu.sync_copy(cdst_hbm.at[pl.ds(go, GCH)],
                        dstb.at[pl.ds(GCH * hb, GCH)])
        for q in range(GCH // 16):
            sl = pl.ds(GCH * hb + q * 16, 16)
            pos = g * GCH + q * 16 + iota16
            mm = pos < cnt
            srcb[sl] = jnp.where(mm, srcb[sl], iota16 & 7)
            dstb[sl] = jnp.where(mm, dstb[sl], TS)

    def start_gather(hb, sem):
        pltpu.async_copy(z_hbm.at[srcb.at[pl.ds(GCH * hb, GCH)]],
                         rows.at[hb], sem)

    def wait_gather(hb, sem):
        pltpu.make_async_copy(z_hbm.at[srcb.at[pl.ds(GCH * hb, GCH)]],
                              rows.at[hb], sem).wait()

    @pl.when(nch > 0)
    def _():
        load_idx(0, 0)
        start_gather(0, sem_a)

    @pl.when(nch > 1)
    def _():
        load_idx(1, 1)
        start_gather(1, sem_b)

    def pair(p, _):
        for hb in range(2):
            g = 2 * p + hb
            sem = sem_a if hb == 0 else sem_b

            @pl.when(g < nch)
            def _():
                wait_gather(hb, sem)

                def acc_e(e, _):
                    sl = dstb[pl.ds(GCH * hb + e, 16)][0]
                    b16 = sl * D + iota16
                    for j in range(lanes):
                        plsc.addupdate_scatter(
                            acc, [b16 + (j * 16)],
                            rows[hb, e, pl.ds(j * 16, 16)])
                    return 0

                lax.fori_loop(0, GCH, acc_e, 0)

                @pl.when(g + 2 < nch)
                def _():
                    load_idx(g + 2, hb)
                    start_gather(hb, sem)

        return 0

    lax.fori_loop(0, (nch + 1) // 2, pair, 0)

    @pl.when(w < NW - 1)
    def _():
        pltpu.sync_copy(acc.at[pl.ds(0, RNG * D)],
                        agg_hbm.at[pl.ds(w * (RNG * D), RNG * D)])

    @pl.when(w == NW - 1)
    def _():
        pltpu.sync_copy(acc.at[pl.ds(0, RNGL * D)],
                        agg_hbm.at[pl.ds(w * (RNG * D), RNGL * D)])


def _mm_body(deg_ref, x_ref, w_ref, z_ref):
    dv = lax.rsqrt(1.0 + deg_ref[...])
    z_ref[...] = jnp.dot(
        x_ref[...] * dv, w_ref[...], preferred_element_type=jnp.float32
    )


def _mm(deg, x, w):
    return pl.pallas_call(
        _mm_body,
        grid=(N // RB,),
        in_specs=[
            pl.BlockSpec((RB, 1), lambda i: (i, 0)),
            pl.BlockSpec((RB, D), lambda i: (i, 0)),
            pl.BlockSpec((D, D), lambda i: (0, 0)),
        ],
        out_specs=pl.BlockSpec((RB, D), lambda i: (i, 0)),
        out_shape=jax.ShapeDtypeStruct((N, D), jnp.float32),
    )(deg, x, w)


def _ln_relu(pre, g_ref, be_ref):
    mu = jnp.mean(pre, axis=-1, keepdims=True)
    xc = pre - mu
    var = jnp.mean(xc * xc, axis=-1, keepdims=True)
    y = xc * lax.rsqrt(var + 1e-5) * g_ref[...] + be_ref[...]
    return jnp.maximum(y, 0.0)


def _ep1_body(deg_ref, agg_ref, z_ref, b_ref, g_ref, be_ref,
              w2_ref, h1_ref, z2_ref):
    dv = lax.rsqrt(1.0 + deg_ref[...])
    pre = dv * (agg_ref[...] + z_ref[...]) + b_ref[...]
    h1 = _ln_relu(pre, g_ref, be_ref)
    h1_ref[...] = h1
    z2_ref[...] = jnp.dot(
        h1 * dv, w2_ref[...], preferred_element_type=jnp.float32
    )


def _ep1(deg, agg1, z1, b1, g1, be1, W2):
    return pl.pallas_call(
        _ep1_body,
        grid=(N // RB,),
        in_specs=[
            pl.BlockSpec((RB, 1), lambda i: (i, 0)),
            pl.BlockSpec((RB, D), lambda i: (i, 0)),
            pl.BlockSpec((RB, D), lambda i: (i, 0)),
            pl.BlockSpec((1, D), lambda i: (0, 0)),
            pl.BlockSpec((1, D), lambda i: (0, 0)),
            pl.BlockSpec((1, D), lambda i: (0, 0)),
            pl.BlockSpec((D, D), lambda i: (0, 0)),
        ],
        out_specs=[
            pl.BlockSpec((RB, D), lambda i: (i, 0)),
            pl.BlockSpec((RB, D), lambda i: (i, 0)),
        ],
        out_shape=[jax.ShapeDtypeStruct((N, D), jnp.float32)] * 2,
    )(deg, agg1, z1, b1, g1, be1, W2)


def _ep2_body(deg_ref, agg_ref, z_ref, b_ref, g_ref, be_ref,
              h1_ref, out_ref):
    dv = lax.rsqrt(1.0 + deg_ref[...])
    pre = dv * (agg_ref[...] + z_ref[...]) + b_ref[...]
    h2 = _ln_relu(pre, g_ref, be_ref)
    out_ref[...] = h1_ref[...] + h2


def _ep2(deg, agg2, z2, b2, g2, be2, h1):
    return pl.pallas_call(
        _ep2_body,
        grid=(N // RB,),
        in_specs=[
            pl.BlockSpec((RB, 1), lambda i: (i, 0)),
            pl.BlockSpec((RB, D), lambda i: (i, 0)),
            pl.BlockSpec((RB, D), lambda i: (i, 0)),
            pl.BlockSpec((1, D), lambda i: (0, 0)),
            pl.BlockSpec((1, D), lambda i: (0, 0)),
            pl.BlockSpec((1, D), lambda i: (0, 0)),
            pl.BlockSpec((RB, D), lambda i: (i, 0)),
        ],
        out_specs=pl.BlockSpec((RB, D), lambda i: (i, 0)),
        out_shape=jax.ShapeDtypeStruct((N, D), jnp.float32),
    )(deg, agg2, z2, b2, g2, be2, h1)


def kernel(x, edge_index, W1, b1, g1, be1, W2, b2, g2, be2):
    src_flat = edge_index[0]
    dst_flat = edge_index[1]

    csrc, cdst, cnts, deg = _compact_kernel(src_flat, dst_flat)
    deg2 = deg.reshape(N, 1)

    z1 = _mm(deg2, x, W1)
    agg1 = _agg_kernel(z1, csrc, cdst, cnts).reshape(N, D)
    h1, z2 = _ep1(deg2, agg1, z1, b1[None], g1[None], be1[None], W2)
    agg2 = _agg_kernel(z2, csrc, cdst, cnts).reshape(N, D)
    return _ep2(deg2, agg2, z2, b2[None], g2[None], be2[None], h1)


# speedup vs baseline: 4.8197x; 1.0009x over previous
import functools

import jax
import jax.numpy as jnp
from jax import lax
from jax.experimental import pallas as pl
from jax.experimental.pallas import tpu as pltpu
from jax.experimental.pallas import tpu_sc as plsc

N = 10000
D = 256
E = 160000

NC = 2
NS = 16
NW = NC * NS
RNG = 312
RNGL = N - (NW - 1) * RNG
ACCR = 336
TS = ACCR - 1
SCH = 2000
NSCH = E // SCH
STG = 4096
FLUSH = 2048
EPW = E + FLUSH
GCH = 64

RB = 1000

_mesh = plsc.VectorSubcoreMesh(
    core_axis_name="c", subcore_axis_name="s", num_cores=NC, num_subcores=NS
)
_params = pltpu.CompilerParams(needs_layout_passes=False)


def _worker(c, t):
    return c * NS + t


@functools.partial(
    pl.kernel,
    out_type=[
        jax.ShapeDtypeStruct((NW * EPW,), jnp.int32),
        jax.ShapeDtypeStruct((NW * EPW,), jnp.int32),
        jax.ShapeDtypeStruct((NW * 16,), jnp.int32),
        jax.ShapeDtypeStruct((N,), jnp.float32),
    ],
    mesh=_mesh,
    compiler_params=_params,
    scratch_types=[
        pltpu.VMEM((SCH,), jnp.int32),
        pltpu.VMEM((SCH,), jnp.int32),
        pltpu.VMEM((STG,), jnp.int32),
        pltpu.VMEM((STG,), jnp.int32),
        pltpu.VMEM((ACCR,), jnp.float32),
    ],
)
def _compact_kernel(src_hbm, dst_hbm, csrc_hbm, cdst_hbm, cnts_hbm, deg_hbm,
                    sbuf, dbuf, stg_s, stg_d, degloc):
    c = lax.axis_index("c")
    t = lax.axis_index("s")
    w = _worker(c, t)
    lo = w * RNG
    hi = jnp.where(w == NW - 1, N, lo + RNG)
    base = w * EPW
    ones16 = jnp.ones((16,), jnp.float32)
    zero16 = jnp.zeros((16,), jnp.float32)

    def zdeg(i, _):
        degloc[pl.ds(i * 16, 16)] = zero16
        return 0

    lax.fori_loop(0, ACCR // 16, zdeg, 0)

    def chunk(k, carry):
        off0, hoff0 = carry
        pltpu.sync_copy(src_hbm.at[pl.ds(k * SCH, SCH)], sbuf)
        pltpu.sync_copy(dst_hbm.at[pl.ds(k * SCH, SCH)], dbuf)

        def grp(j, off):
            d16 = dbuf[pl.ds(j * 16, 16)]
            s16 = sbuf[pl.ds(j * 16, 16)]
            dl = d16 - lo
            m = (d16 >= lo) & (d16 < hi)
            plsc.store_compressed(stg_s.at[pl.ds(off, 16)], s16, mask=m)
            plsc.store_compressed(stg_d.at[pl.ds(off, 16)], dl, mask=m)
            plsc.addupdate_scatter(
                degloc, [jnp.where(m, dl, TS)], ones16, mask=m)
            pc = plsc.all_reduce_population_count(m)
            return off + pc[0]

        off = lax.fori_loop(0, SCH // 16, grp, off0)

        def do_flush(a):
            o, h = a
            ho = pl.multiple_of(base + h, 8)
            pltpu.sync_copy(stg_s.at[pl.ds(0, FLUSH)],
                            csrc_hbm.at[pl.ds(ho, FLUSH)])
            pltpu.sync_copy(stg_d.at[pl.ds(0, FLUSH)],
                            cdst_hbm.at[pl.ds(ho, FLUSH)])

            def shift(i, _):
                stg_s[pl.ds(i * 16, 16)] = stg_s[pl.ds(FLUSH + i * 16, 16)]
                stg_d[pl.ds(i * 16, 16)] = stg_d[pl.ds(FLUSH + i * 16, 16)]
                return 0

            lax.fori_loop(0, FLUSH // 16, shift, 0)
            return (o - FLUSH, h + FLUSH)

        return lax.cond(off >= FLUSH, do_flush, lambda a: a, (off, hoff0))

    off, hoff = lax.fori_loop(0, NSCH, chunk, (jnp.int32(0), jnp.int32(0)))

    hof = pl.multiple_of(base + hoff, 8)
    pltpu.sync_copy(stg_s.at[pl.ds(0, FLUSH)],
                    csrc_hbm.at[pl.ds(hof, FLUSH)])
    pltpu.sync_copy(stg_d.at[pl.ds(0, FLUSH)],
                    cdst_hbm.at[pl.ds(hof, FLUSH)])

    sbuf[pl.ds(0, 16)] = jnp.broadcast_to(hoff + off, (16,))
    pltpu.sync_copy(sbuf.at[pl.ds(0, 16)], cnts_hbm.at[pl.ds(w * 16, 16)])

    @pl.when(w < NW - 1)
    def _():
        pltpu.sync_copy(degloc.at[pl.ds(0, RNG)],
                        deg_hbm.at[pl.ds(lo, RNG)])

    @pl.when(w == NW - 1)
    def _():
        pltpu.sync_copy(degloc.at[pl.ds(0, RNGL)],
                        deg_hbm.at[pl.ds(lo, RNGL)])


@functools.partial(
    pl.kernel,
    out_type=jax.ShapeDtypeStruct((N * D,), jnp.float32),
    mesh=_mesh,
    compiler_params=_params,
    scratch_types=[
        pltpu.VMEM((160,), jnp.int32),
        pltpu.VMEM((160,), jnp.int32),
        pltpu.VMEM((2, GCH, D), jnp.float32),
        pltpu.VMEM((ACCR * D,), jnp.float32),
        pltpu.SemaphoreType.DMA,
        pltpu.SemaphoreType.DMA,
    ],
)
def _agg_kernel(z_hbm, csrc_hbm, cdst_hbm, cnts_hbm, agg_hbm,
                srcb, dstb, rows, acc, sem_a, sem_b):
    c = lax.axis_index("c")
    t = lax.axis_index("s")
    w = _worker(c, t)
    base = w * EPW
    iota16 = lax.iota(jnp.int32, 16)
    zero16 = jnp.zeros((16,), jnp.float32)
    lanes = D // 16

    pltpu.sync_copy(cnts_hbm.at[pl.ds(w * 16, 16)], srcb.at[pl.ds(0, 16)])
    cnt = srcb[pl.ds(0, 16)][0]
    nch = (cnt + (GCH - 1)) // GCH

    def zacc(i, _):
        acc[pl.ds(i * 16, 16)] = zero16
        return 0

    lax.fori_loop(0, ACCR * lanes, zacc, 0)

    def load_idx(g, hb):
        go = pl.multiple_of(base + g * GCH, 8)
        pltpu.sync_copy(csrc_hbm.at[pl.ds(go, GCH)],
                        srcb.at[pl.ds(GCH * hb, GCH)])
        pltpu.sync_copy(cdst_hbm.at[pl.ds(go, GCH)],
                        dstb.at[pl.ds(GCH * hb, GCH)])
        for q in range(GCH // 16):
            sl = pl.ds(GCH * hb + q * 16, 16)
            pos = g * GCH + q * 16 + iota16
            mm = pos < cnt
            srcb[sl] = jnp.where(mm, srcb[sl], iota16 & 7)
            dstb[sl] = jnp.where(mm, dstb[sl], TS)

    def start_gather(hb, sem):
        pltpu.async_copy(z_hbm.at[srcb.at[pl.ds(GCH * hb, GCH)]],
                         rows.at[hb], sem)

    def wait_gather(hb, sem):
        pltpu.make_async_copy(z_hbm.at[srcb.at[pl.ds(GCH * hb, GCH)]],
                              rows.at[hb], sem).wait()

    @pl.when(nch > 0)
    def _():
        load_idx(0, 0)
        start_gather(0, sem_a)

    @pl.when(nch > 1)
    def _():
        load_idx(1, 1)
        start_gather(1, sem_b)

    def pair(p, _):
        for hb in range(2):
            g = 2 * p + hb
            sem = sem_a if hb == 0 else sem_b

            @pl.when(g < nch)
            def _():
                wait_gather(hb, sem)

                def acc_q(q, _):
                    for e in range(16):
                        sl = dstb[pl.ds(GCH * hb + q * 16 + e, 16)][0]
                        be = sl * D + iota16
                        for j in range(lanes):
                            plsc.addupdate_scatter(
                                acc, [be + (j * 16)],
                                rows[hb, q * 16 + e, pl.ds(j * 16, 16)])
                    return 0

                lax.fori_loop(0, GCH // 16, acc_q, 0)

                @pl.when(g + 2 < nch)
                def _():
                    load_idx(g + 2, hb)
                    start_gather(hb, sem)

        return 0

    lax.fori_loop(0, (nch + 1) // 2, pair, 0)

    @pl.when(w < NW - 1)
    def _():
        pltpu.sync_copy(acc.at[pl.ds(0, RNG * D)],
                        agg_hbm.at[pl.ds(w * (RNG * D), RNG * D)])

    @pl.when(w == NW - 1)
    def _():
        pltpu.sync_copy(acc.at[pl.ds(0, RNGL * D)],
                        agg_hbm.at[pl.ds(w * (RNG * D), RNGL * D)])


def _mm_body(deg_ref, x_ref, w_ref, z_ref):
    dv = lax.rsqrt(1.0 + deg_ref[...])
    z_ref[...] = jnp.dot(
        x_ref[...] * dv, w_ref[...], preferred_element_type=jnp.float32
    )


def _mm(deg, x, w):
    return pl.pallas_call(
        _mm_body,
        grid=(N // RB,),
        in_specs=[
            pl.BlockSpec((RB, 1), lambda i: (i, 0)),
            pl.BlockSpec((RB, D), lambda i: (i, 0)),
            pl.BlockSpec((D, D), lambda i: (0, 0)),
        ],
        out_specs=pl.BlockSpec((RB, D), lambda i: (i, 0)),
        out_shape=jax.ShapeDtypeStruct((N, D), jnp.float32),
    )(deg, x, w)


def _ln_relu(pre, g_ref, be_ref):
    mu = jnp.mean(pre, axis=-1, keepdims=True)
    xc = pre - mu
    var = jnp.mean(xc * xc, axis=-1, keepdims=True)
    y = xc * lax.rsqrt(var + 1e-5) * g_ref[...] + be_ref[...]
    return jnp.maximum(y, 0.0)


def _ep1_body(deg_ref, agg_ref, z_ref, b_ref, g_ref, be_ref,
              w2_ref, h1_ref, z2_ref):
    dv = lax.rsqrt(1.0 + deg_ref[...])
    pre = dv * (agg_ref[...] + z_ref[...]) + b_ref[...]
    h1 = _ln_relu(pre, g_ref, be_ref)
    h1_ref[...] = h1
    z2_ref[...] = jnp.dot(
        h1 * dv, w2_ref[...], preferred_element_type=jnp.float32
    )


def _ep1(deg, agg1, z1, b1, g1, be1, W2):
    return pl.pallas_call(
        _ep1_body,
        grid=(N // RB,),
        in_specs=[
            pl.BlockSpec((RB, 1), lambda i: (i, 0)),
            pl.BlockSpec((RB, D), lambda i: (i, 0)),
            pl.BlockSpec((RB, D), lambda i: (i, 0)),
            pl.BlockSpec((1, D), lambda i: (0, 0)),
            pl.BlockSpec((1, D), lambda i: (0, 0)),
            pl.BlockSpec((1, D), lambda i: (0, 0)),
            pl.BlockSpec((D, D), lambda i: (0, 0)),
        ],
        out_specs=[
            pl.BlockSpec((RB, D), lambda i: (i, 0)),
            pl.BlockSpec((RB, D), lambda i: (i, 0)),
        ],
        out_shape=[jax.ShapeDtypeStruct((N, D), jnp.float32)] * 2,
    )(deg, agg1, z1, b1, g1, be1, W2)


def _ep2_body(deg_ref, agg_ref, z_ref, b_ref, g_ref, be_ref,
              h1_ref, out_ref):
    dv = lax.rsqrt(1.0 + deg_ref[...])
    pre = dv * (agg_ref[...] + z_ref[...]) + b_ref[...]
    h2 = _ln_relu(pre, g_ref, be_ref)
    out_ref[...] = h1_ref[...] + h2


def _ep2(deg, agg2, z2, b2, g2, be2, h1):
    return pl.pallas_call(
        _ep2_body,
        grid=(N // RB,),
        in_specs=[
            pl.BlockSpec((RB, 1), lambda i: (i, 0)),
            pl.BlockSpec((RB, D), lambda i: (i, 0)),
            pl.BlockSpec((RB, D), lambda i: (i, 0)),
            pl.BlockSpec((1, D), lambda i: (0, 0)),
            pl.BlockSpec((1, D), lambda i: (0, 0)),
            pl.BlockSpec((1, D), lambda i: (0, 0)),
            pl.BlockSpec((RB, D), lambda i: (i, 0)),
        ],
        out_specs=pl.BlockSpec((RB, D), lambda i: (i, 0)),
        out_shape=jax.ShapeDtypeStruct((N, D), jnp.float32),
    )(deg, agg2, z2, b2, g2, be2, h1)


def kernel(x, edge_index, W1, b1, g1, be1, W2, b2, g2, be2):
    src_flat = edge_index[0]
    dst_flat = edge_index[1]

    csrc, cdst, cnts, deg = _compact_kernel(src_flat, dst_flat)
    deg2 = deg.reshape(N, 1)

    z1 = _mm(deg2, x, W1)
    agg1 = _agg_kernel(z1, csrc, cdst, cnts).reshape(N, D)
    h1, z2 = _ep1(deg2, agg1, z1, b1[None], g1[None], be1[None], W2)
    agg2 = _agg_kernel(z2, csrc, cdst, cnts).reshape(N, D)
    return _ep2(deg2, agg2, z2, b2[None], g2[None], be2[None], h1)


# speedup vs baseline: 5.3843x; 1.1172x over previous
import functools

import jax
import jax.numpy as jnp
from jax import lax
from jax.experimental import pallas as pl
from jax.experimental.pallas import tpu as pltpu
from jax.experimental.pallas import tpu_sc as plsc

N = 10000
D = 256
E = 160000

NC = 2
NS = 16
NW = NC * NS
RNG = 312
RNGL = N - (NW - 1) * RNG
ACCR = 336
TS = ACCR - 1
SCH = 2000
NSCH = E // SCH
STG = 4096
FLUSH = 2048
EPW = E + FLUSH
GCH = 64

RB = 1000

_mesh = plsc.VectorSubcoreMesh(
    core_axis_name="c", subcore_axis_name="s", num_cores=NC, num_subcores=NS
)
_params = pltpu.CompilerParams(needs_layout_passes=False)


def _worker(c, t):
    return c * NS + t


@functools.partial(
    pl.kernel,
    out_type=[
        jax.ShapeDtypeStruct((NW * EPW,), jnp.int32),
        jax.ShapeDtypeStruct((NW * EPW,), jnp.int32),
        jax.ShapeDtypeStruct((NW * 16,), jnp.int32),
        jax.ShapeDtypeStruct((N,), jnp.float32),
    ],
    mesh=_mesh,
    compiler_params=_params,
    scratch_types=[
        pltpu.VMEM((SCH,), jnp.int32),
        pltpu.VMEM((SCH,), jnp.int32),
        pltpu.VMEM((STG,), jnp.int32),
        pltpu.VMEM((STG,), jnp.int32),
        pltpu.VMEM((ACCR,), jnp.float32),
    ],
)
def _compact_kernel(src_hbm, dst_hbm, csrc_hbm, cdst_hbm, cnts_hbm, deg_hbm,
                    sbuf, dbuf, stg_s, stg_d, degloc):
    c = lax.axis_index("c")
    t = lax.axis_index("s")
    w = _worker(c, t)
    lo = w * RNG
    hi = jnp.where(w == NW - 1, N, lo + RNG)
    base = w * EPW
    ones16 = jnp.ones((16,), jnp.float32)
    zero16 = jnp.zeros((16,), jnp.float32)

    def zdeg(i, _):
        degloc[pl.ds(i * 16, 16)] = zero16
        return 0

    lax.fori_loop(0, ACCR // 16, zdeg, 0)

    def chunk(k, carry):
        off0, hoff0 = carry
        pltpu.sync_copy(src_hbm.at[pl.ds(k * SCH, SCH)], sbuf)
        pltpu.sync_copy(dst_hbm.at[pl.ds(k * SCH, SCH)], dbuf)

        def grp(j, off):
            d16 = dbuf[pl.ds(j * 16, 16)]
            s16 = sbuf[pl.ds(j * 16, 16)]
            dl = d16 - lo
            m = (d16 >= lo) & (d16 < hi)
            plsc.store_compressed(stg_s.at[pl.ds(off, 16)], s16, mask=m)
            plsc.store_compressed(stg_d.at[pl.ds(off, 16)], dl, mask=m)
            plsc.addupdate_scatter(
                degloc, [jnp.where(m, dl, TS)], ones16, mask=m)
            pc = plsc.all_reduce_population_count(m)
            return off + pc[0]

        off = lax.fori_loop(0, SCH // 16, grp, off0)

        def do_flush(a):
            o, h = a
            ho = pl.multiple_of(base + h, 8)
            pltpu.sync_copy(stg_s.at[pl.ds(0, FLUSH)],
                            csrc_hbm.at[pl.ds(ho, FLUSH)])
            pltpu.sync_copy(stg_d.at[pl.ds(0, FLUSH)],
                            cdst_hbm.at[pl.ds(ho, FLUSH)])

            def shift(i, _):
                stg_s[pl.ds(i * 16, 16)] = stg_s[pl.ds(FLUSH + i * 16, 16)]
                stg_d[pl.ds(i * 16, 16)] = stg_d[pl.ds(FLUSH + i * 16, 16)]
                return 0

            lax.fori_loop(0, FLUSH // 16, shift, 0)
            return (o - FLUSH, h + FLUSH)

        return lax.cond(off >= FLUSH, do_flush, lambda a: a, (off, hoff0))

    off, hoff = lax.fori_loop(0, NSCH, chunk, (jnp.int32(0), jnp.int32(0)))

    hof = pl.multiple_of(base + hoff, 8)
    pltpu.sync_copy(stg_s.at[pl.ds(0, FLUSH)],
                    csrc_hbm.at[pl.ds(hof, FLUSH)])
    pltpu.sync_copy(stg_d.at[pl.ds(0, FLUSH)],
                    cdst_hbm.at[pl.ds(hof, FLUSH)])

    sbuf[pl.ds(0, 16)] = jnp.broadcast_to(hoff + off, (16,))
    pltpu.sync_copy(sbuf.at[pl.ds(0, 16)], cnts_hbm.at[pl.ds(w * 16, 16)])

    @pl.when(w < NW - 1)
    def _():
        pltpu.sync_copy(degloc.at[pl.ds(0, RNG)],
                        deg_hbm.at[pl.ds(lo, RNG)])

    @pl.when(w == NW - 1)
    def _():
        pltpu.sync_copy(degloc.at[pl.ds(0, RNGL)],
                        deg_hbm.at[pl.ds(lo, RNGL)])


@functools.partial(
    pl.kernel,
    out_type=jax.ShapeDtypeStruct((N * D,), jnp.float32),
    mesh=_mesh,
    compiler_params=_params,
    scratch_types=[
        pltpu.VMEM((160,), jnp.int32),
        pltpu.VMEM((160,), jnp.int32),
        pltpu.VMEM((2, GCH, D), jnp.float32),
        pltpu.VMEM((ACCR * D,), jnp.float32),
        pltpu.SemaphoreType.DMA,
        pltpu.SemaphoreType.DMA,
    ],
)
def _agg_kernel(z_hbm, csrc_hbm, cdst_hbm, cnts_hbm, agg_hbm,
                srcb, dstb, rows, acc, sem_a, sem_b):
    c = lax.axis_index("c")
    t = lax.axis_index("s")
    w = _worker(c, t)
    base = w * EPW
    iota16 = lax.iota(jnp.int32, 16)
    zero16 = jnp.zeros((16,), jnp.float32)
    lanes = D // 16

    pltpu.sync_copy(cnts_hbm.at[pl.ds(w * 16, 16)], srcb.at[pl.ds(0, 16)])
    cnt = srcb[pl.ds(0, 16)][0]
    nch = (cnt + (GCH - 1)) // GCH

    def zacc(i, _):
        acc[pl.ds(i * 16, 16)] = zero16
        return 0

    lax.fori_loop(0, ACCR * lanes, zacc, 0)

    def load_idx(g, hb):
        go = pl.multiple_of(base + g * GCH, 8)
        pltpu.sync_copy(csrc_hbm.at[pl.ds(go, GCH)],
                        srcb.at[pl.ds(GCH * hb, GCH)])
        pltpu.sync_copy(cdst_hbm.at[pl.ds(go, GCH)],
                        dstb.at[pl.ds(GCH * hb, GCH)])
        for q in range(GCH // 16):
            sl = pl.ds(GCH * hb + q * 16, 16)
            pos = g * GCH + q * 16 + iota16
            mm = pos < cnt
            srcb[sl] = jnp.where(mm, srcb[sl], iota16 & 7)
            dstb[sl] = jnp.where(mm, dstb[sl], TS)

    def start_gather(hb, sem):
        pltpu.async_copy(z_hbm.at[srcb.at[pl.ds(GCH * hb, GCH)]],
                         rows.at[hb], sem)

    def wait_gather(hb, sem):
        pltpu.make_async_copy(z_hbm.at[srcb.at[pl.ds(GCH * hb, GCH)]],
                              rows.at[hb], sem).wait()

    @pl.when(nch > 0)
    def _():
        load_idx(0, 0)
        start_gather(0, sem_a)

    @pl.when(nch > 1)
    def _():
        load_idx(1, 1)
        start_gather(1, sem_b)

    def pair(p, _):
        for hb in range(2):
            g = 2 * p + hb
            sem = sem_a if hb == 0 else sem_b

            @pl.when(g < nch)
            def _():
                wait_gather(hb, sem)

                def acc_q(q, _):
                    slots16 = dstb[pl.ds(GCH * hb + q * 16, 16)]
                    for e in range(16):
                        sl = slots16[e]
                        be = sl * D + iota16
                        for j in range(lanes):
                            plsc.addupdate_scatter(
                                acc, [be + (j * 16)],
                                rows[hb, q * 16 + e, pl.ds(j * 16, 16)])
                    return 0

                lax.fori_loop(0, GCH // 16, acc_q, 0)

                @pl.when(g + 2 < nch)
                def _():
                    load_idx(g + 2, hb)
                    start_gather(hb, sem)

        return 0

    lax.fori_loop(0, (nch + 1) // 2, pair, 0)

    @pl.when(w < NW - 1)
    def _():
        pltpu.sync_copy(acc.at[pl.ds(0, RNG * D)],
                        agg_hbm.at[pl.ds(w * (RNG * D), RNG * D)])

    @pl.when(w == NW - 1)
    def _():
        pltpu.sync_copy(acc.at[pl.ds(0, RNGL * D)],
                        agg_hbm.at[pl.ds(w * (RNG * D), RNGL * D)])


def _mm_body(deg_ref, x_ref, w_ref, z_ref):
    dv = lax.rsqrt(1.0 + deg_ref[...])
    z_ref[...] = jnp.dot(
        x_ref[...] * dv, w_ref[...], preferred_element_type=jnp.float32
    )


def _mm(deg, x, w):
    return pl.pallas_call(
        _mm_body,
        grid=(N // RB,),
        in_specs=[
            pl.BlockSpec((RB, 1), lambda i: (i, 0)),
            pl.BlockSpec((RB, D), lambda i: (i, 0)),
            pl.BlockSpec((D, D), lambda i: (0, 0)),
        ],
        out_specs=pl.BlockSpec((RB, D), lambda i: (i, 0)),
        out_shape=jax.ShapeDtypeStruct((N, D), jnp.float32),
    )(deg, x, w)


def _ln_relu(pre, g_ref, be_ref):
    mu = jnp.mean(pre, axis=-1, keepdims=True)
    xc = pre - mu
    var = jnp.mean(xc * xc, axis=-1, keepdims=True)
    y = xc * lax.rsqrt(var + 1e-5) * g_ref[...] + be_ref[...]
    return jnp.maximum(y, 0.0)


def _ep1_body(deg_ref, agg_ref, z_ref, b_ref, g_ref, be_ref,
              w2_ref, h1_ref, z2_ref):
    dv = lax.rsqrt(1.0 + deg_ref[...])
    pre = dv * (agg_ref[...] + z_ref[...]) + b_ref[...]
    h1 = _ln_relu(pre, g_ref, be_ref)
    h1_ref[...] = h1
    z2_ref[...] = jnp.dot(
        h1 * dv, w2_ref[...], preferred_element_type=jnp.float32
    )


def _ep1(deg, agg1, z1, b1, g1, be1, W2):
    return pl.pallas_call(
        _ep1_body,
        grid=(N // RB,),
        in_specs=[
            pl.BlockSpec((RB, 1), lambda i: (i, 0)),
            pl.BlockSpec((RB, D), lambda i: (i, 0)),
            pl.BlockSpec((RB, D), lambda i: (i, 0)),
            pl.BlockSpec((1, D), lambda i: (0, 0)),
            pl.BlockSpec((1, D), lambda i: (0, 0)),
            pl.BlockSpec((1, D), lambda i: (0, 0)),
            pl.BlockSpec((D, D), lambda i: (0, 0)),
        ],
        out_specs=[
            pl.BlockSpec((RB, D), lambda i: (i, 0)),
            pl.BlockSpec((RB, D), lambda i: (i, 0)),
        ],
        out_shape=[jax.ShapeDtypeStruct((N, D), jnp.float32)] * 2,
    )(deg, agg1, z1, b1, g1, be1, W2)


def _ep2_body(deg_ref, agg_ref, z_ref, b_ref, g_ref, be_ref,
              h1_ref, out_ref):
    dv = lax.rsqrt(1.0 + deg_ref[...])
    pre = dv * (agg_ref[...] + z_ref[...]) + b_ref[...]
    h2 = _ln_relu(pre, g_ref, be_ref)
    out_ref[...] = h1_ref[...] + h2


def _ep2(deg, agg2, z2, b2, g2, be2, h1):
    return pl.pallas_call(
        _ep2_body,
        grid=(N // RB,),
        in_specs=[
            pl.BlockSpec((RB, 1), lambda i: (i, 0)),
            pl.BlockSpec((RB, D), lambda i: (i, 0)),
            pl.BlockSpec((RB, D), lambda i: (i, 0)),
            pl.BlockSpec((1, D), lambda i: (0, 0)),
            pl.BlockSpec((1, D), lambda i: (0, 0)),
            pl.BlockSpec((1, D), lambda i: (0, 0)),
            pl.BlockSpec((RB, D), lambda i: (i, 0)),
        ],
        out_specs=pl.BlockSpec((RB, D), lambda i: (i, 0)),
        out_shape=jax.ShapeDtypeStruct((N, D), jnp.float32),
    )(deg, agg2, z2, b2, g2, be2, h1)


def kernel(x, edge_index, W1, b1, g1, be1, W2, b2, g2, be2):
    src_flat = edge_index[0]
    dst_flat = edge_index[1]

    csrc, cdst, cnts, deg = _compact_kernel(src_flat, dst_flat)
    deg2 = deg.reshape(N, 1)

    z1 = _mm(deg2, x, W1)
    agg1 = _agg_kernel(z1, csrc, cdst, cnts).reshape(N, D)
    h1, z2 = _ep1(deg2, agg1, z1, b1[None], g1[None], be1[None], W2)
    agg2 = _agg_kernel(z2, csrc, cdst, cnts).reshape(N, D)
    return _ep2(deg2, agg2, z2, b2[None], g2[None], be2[None], h1)


# speedup vs baseline: 5.9857x; 1.1117x over previous
import functools

import jax
import jax.numpy as jnp
from jax import lax
from jax.experimental import pallas as pl
from jax.experimental.pallas import tpu as pltpu
from jax.experimental.pallas import tpu_sc as plsc

N = 10000
D = 256
E = 160000

NC = 2
NS = 16
NW = NC * NS
RNG = 312
RNGL = N - (NW - 1) * RNG
ACCR = 336
TS = ACCR - 1
SCH = 2000
NSCH = E // SCH
STG = 4096
FLUSH = 2048
EPW = E + FLUSH
GCH = 64

RB = 1000

_mesh = plsc.VectorSubcoreMesh(
    core_axis_name="c", subcore_axis_name="s", num_cores=NC, num_subcores=NS
)
_params = pltpu.CompilerParams(needs_layout_passes=False)


def _worker(c, t):
    return c * NS + t


@functools.partial(
    pl.kernel,
    out_type=[
        jax.ShapeDtypeStruct((NW * EPW,), jnp.int32),
        jax.ShapeDtypeStruct((NW * EPW,), jnp.int32),
        jax.ShapeDtypeStruct((NW * 16,), jnp.int32),
        jax.ShapeDtypeStruct((N,), jnp.float32),
    ],
    mesh=_mesh,
    compiler_params=_params,
    scratch_types=[
        pltpu.VMEM((SCH,), jnp.int32),
        pltpu.VMEM((SCH,), jnp.int32),
        pltpu.VMEM((STG,), jnp.int32),
        pltpu.VMEM((STG,), jnp.int32),
        pltpu.VMEM((ACCR,), jnp.float32),
    ],
)
def _compact_kernel(src_hbm, dst_hbm, csrc_hbm, cdst_hbm, cnts_hbm, deg_hbm,
                    sbuf, dbuf, stg_s, stg_d, degloc):
    c = lax.axis_index("c")
    t = lax.axis_index("s")
    w = _worker(c, t)
    lo = w * RNG
    hi = jnp.where(w == NW - 1, N, lo + RNG)
    base = w * EPW
    ones16 = jnp.ones((16,), jnp.float32)
    zero16 = jnp.zeros((16,), jnp.float32)

    def zdeg(i, _):
        degloc[pl.ds(i * 16, 16)] = zero16
        return 0

    lax.fori_loop(0, ACCR // 16, zdeg, 0)

    def chunk(k, carry):
        off0, hoff0 = carry
        pltpu.sync_copy(src_hbm.at[pl.ds(k * SCH, SCH)], sbuf)
        pltpu.sync_copy(dst_hbm.at[pl.ds(k * SCH, SCH)], dbuf)

        def grp(j, off):
            d16 = dbuf[pl.ds(j * 16, 16)]
            s16 = sbuf[pl.ds(j * 16, 16)]
            dl = d16 - lo
            m = (d16 >= lo) & (d16 < hi)
            plsc.store_compressed(stg_s.at[pl.ds(off, 16)], s16, mask=m)
            plsc.store_compressed(stg_d.at[pl.ds(off, 16)], dl, mask=m)
            plsc.addupdate_scatter(
                degloc, [jnp.where(m, dl, TS)], ones16, mask=m)
            pc = plsc.all_reduce_population_count(m)
            return off + pc[0]

        off = lax.fori_loop(0, SCH // 16, grp, off0)

        def do_flush(a):
            o, h = a
            ho = pl.multiple_of(base + h, 8)
            pltpu.sync_copy(stg_s.at[pl.ds(0, FLUSH)],
                            csrc_hbm.at[pl.ds(ho, FLUSH)])
            pltpu.sync_copy(stg_d.at[pl.ds(0, FLUSH)],
                            cdst_hbm.at[pl.ds(ho, FLUSH)])

            def shift(i, _):
                stg_s[pl.ds(i * 16, 16)] = stg_s[pl.ds(FLUSH + i * 16, 16)]
                stg_d[pl.ds(i * 16, 16)] = stg_d[pl.ds(FLUSH + i * 16, 16)]
                return 0

            lax.fori_loop(0, FLUSH // 16, shift, 0)
            return (o - FLUSH, h + FLUSH)

        return lax.cond(off >= FLUSH, do_flush, lambda a: a, (off, hoff0))

    off, hoff = lax.fori_loop(0, NSCH, chunk, (jnp.int32(0), jnp.int32(0)))

    hof = pl.multiple_of(base + hoff, 8)
    pltpu.sync_copy(stg_s.at[pl.ds(0, FLUSH)],
                    csrc_hbm.at[pl.ds(hof, FLUSH)])
    pltpu.sync_copy(stg_d.at[pl.ds(0, FLUSH)],
                    cdst_hbm.at[pl.ds(hof, FLUSH)])

    sbuf[pl.ds(0, 16)] = jnp.broadcast_to(hoff + off, (16,))
    pltpu.sync_copy(sbuf.at[pl.ds(0, 16)], cnts_hbm.at[pl.ds(w * 16, 16)])

    @pl.when(w < NW - 1)
    def _():
        pltpu.sync_copy(degloc.at[pl.ds(0, RNG)],
                        deg_hbm.at[pl.ds(lo, RNG)])

    @pl.when(w == NW - 1)
    def _():
        pltpu.sync_copy(degloc.at[pl.ds(0, RNGL)],
                        deg_hbm.at[pl.ds(lo, RNGL)])


@functools.partial(
    pl.kernel,
    out_type=jax.ShapeDtypeStruct((N * D,), jnp.float32),
    mesh=_mesh,
    compiler_params=_params,
    scratch_types=[
        pltpu.VMEM((160,), jnp.int32),
        pltpu.VMEM((272,), jnp.int32),
        pltpu.VMEM((2, GCH, D), jnp.float32),
        pltpu.VMEM((ACCR * D,), jnp.float32),
        pltpu.SemaphoreType.DMA,
        pltpu.SemaphoreType.DMA,
        pltpu.SemaphoreType.DMA,
    ],
)
def _agg_kernel(z_hbm, csrc_hbm, cdst_hbm, cnts_hbm, agg_hbm,
                srcb, dstb, rows, acc, sem_a, sem_b, sem_i):
    c = lax.axis_index("c")
    t = lax.axis_index("s")
    w = _worker(c, t)
    base = w * EPW
    iota16 = lax.iota(jnp.int32, 16)
    zero16 = jnp.zeros((16,), jnp.float32)
    lanes = D // 16

    pltpu.sync_copy(cnts_hbm.at[pl.ds(w * 16, 16)], srcb.at[pl.ds(0, 16)])
    cnt = srcb[pl.ds(0, 16)][0]
    nch = (cnt + (GCH - 1)) // GCH

    def zacc(i, _):
        acc[pl.ds(i * 16, 16)] = zero16
        return 0

    lax.fori_loop(0, ACCR * lanes, zacc, 0)

    def fetch_idx(g, sb, db, sync):
        go = pl.multiple_of(base + g * GCH, 8)
        if sync:
            pltpu.sync_copy(csrc_hbm.at[pl.ds(go, GCH)],
                            srcb.at[pl.ds(GCH * sb, GCH)])
            pltpu.sync_copy(cdst_hbm.at[pl.ds(go, GCH)],
                            dstb.at[pl.ds(GCH * db, GCH)])
        else:
            pltpu.async_copy(csrc_hbm.at[pl.ds(go, GCH)],
                             srcb.at[pl.ds(GCH * sb, GCH)], sem_i)
            pltpu.async_copy(cdst_hbm.at[pl.ds(go, GCH)],
                             dstb.at[pl.ds(GCH * db, GCH)], sem_i)

    def wait_idx(sb, db):
        pltpu.make_async_copy(csrc_hbm.at[pl.ds(base, GCH)],
                              srcb.at[pl.ds(GCH * sb, GCH)], sem_i).wait()
        pltpu.make_async_copy(cdst_hbm.at[pl.ds(base, GCH)],
                              dstb.at[pl.ds(GCH * db, GCH)], sem_i).wait()

    def remap(g, sb, db):
        for q in range(GCH // 16):
            ssl = pl.ds(GCH * sb + q * 16, 16)
            dsl = pl.ds(GCH * db + q * 16, 16)
            pos = g * GCH + q * 16 + iota16
            mm = pos < cnt
            srcb[ssl] = jnp.where(mm, srcb[ssl], iota16 & 7)
            dstb[dsl] = jnp.where(mm, dstb[dsl], TS)

    def start_gather(sb, sem):
        pltpu.async_copy(z_hbm.at[srcb.at[pl.ds(GCH * sb, GCH)]],
                         rows.at[sb], sem)

    def wait_gather(sb, sem):
        pltpu.make_async_copy(z_hbm.at[srcb.at[pl.ds(GCH * sb, GCH)]],
                              rows.at[sb], sem).wait()

    def acc_chunk(sb, db):
        def acc_q(q, _):
            slots16 = dstb[pl.ds(GCH * db + q * 16, 16)]
            for e in range(16):
                sl = slots16[e]
                be = sl * D + iota16
                for j in range(lanes):
                    plsc.addupdate_scatter(
                        acc, [be + (j * 16)],
                        rows[sb, q * 16 + e, pl.ds(j * 16, 16)])
            return 0

        lax.fori_loop(0, GCH // 16, acc_q, 0)

    @pl.when(nch > 0)
    def _():
        fetch_idx(0, 0, 0, True)
        remap(0, 0, 0)
        start_gather(0, sem_a)

    @pl.when(nch > 1)
    def _():
        fetch_idx(1, 1, 1, True)
        remap(1, 1, 1)
        start_gather(1, sem_b)

    def quad(p4, _):
        for k in range(4):
            g = 4 * p4 + k
            sb = k % 2
            db = k
            nd = (k + 2) % 4
            sem = sem_a if sb == 0 else sem_b

            @pl.when(g < nch)
            def _():
                wait_gather(sb, sem)

                @pl.when(g + 2 < nch)
                def _():
                    fetch_idx(g + 2, sb, nd, False)

                acc_chunk(sb, db)

                @pl.when(g + 2 < nch)
                def _():
                    wait_idx(sb, nd)
                    remap(g + 2, sb, nd)
                    start_gather(sb, sem)

        return 0

    lax.fori_loop(0, (nch + 3) // 4, quad, 0)

    @pl.when(w < NW - 1)
    def _():
        pltpu.sync_copy(acc.at[pl.ds(0, RNG * D)],
                        agg_hbm.at[pl.ds(w * (RNG * D), RNG * D)])

    @pl.when(w == NW - 1)
    def _():
        pltpu.sync_copy(acc.at[pl.ds(0, RNGL * D)],
                        agg_hbm.at[pl.ds(w * (RNG * D), RNGL * D)])


def _mm_body(deg_ref, x_ref, w_ref, z_ref):
    dv = lax.rsqrt(1.0 + deg_ref[...])
    z_ref[...] = jnp.dot(
        x_ref[...] * dv, w_ref[...], preferred_element_type=jnp.float32
    )


def _mm(deg, x, w):
    return pl.pallas_call(
        _mm_body,
        grid=(N // RB,),
        in_specs=[
            pl.BlockSpec((RB, 1), lambda i: (i, 0)),
            pl.BlockSpec((RB, D), lambda i: (i, 0)),
            pl.BlockSpec((D, D), lambda i: (0, 0)),
        ],
        out_specs=pl.BlockSpec((RB, D), lambda i: (i, 0)),
        out_shape=jax.ShapeDtypeStruct((N, D), jnp.float32),
    )(deg, x, w)


def _ln_relu(pre, g_ref, be_ref):
    mu = jnp.mean(pre, axis=-1, keepdims=True)
    xc = pre - mu
    var = jnp.mean(xc * xc, axis=-1, keepdims=True)
    y = xc * lax.rsqrt(var + 1e-5) * g_ref[...] + be_ref[...]
    return jnp.maximum(y, 0.0)


def _ep1_body(deg_ref, agg_ref, z_ref, b_ref, g_ref, be_ref,
              w2_ref, h1_ref, z2_ref):
    dv = lax.rsqrt(1.0 + deg_ref[...])
    pre = dv * (agg_ref[...] + z_ref[...]) + b_ref[...]
    h1 = _ln_relu(pre, g_ref, be_ref)
    h1_ref[...] = h1
    z2_ref[...] = jnp.dot(
        h1 * dv, w2_ref[...], preferred_element_type=jnp.float32
    )


def _ep1(deg, agg1, z1, b1, g1, be1, W2):
    return pl.pallas_call(
        _ep1_body,
        grid=(N // RB,),
        in_specs=[
            pl.BlockSpec((RB, 1), lambda i: (i, 0)),
            pl.BlockSpec((RB, D), lambda i: (i, 0)),
            pl.BlockSpec((RB, D), lambda i: (i, 0)),
            pl.BlockSpec((1, D), lambda i: (0, 0)),
            pl.BlockSpec((1, D), lambda i: (0, 0)),
            pl.BlockSpec((1, D), lambda i: (0, 0)),
            pl.BlockSpec((D, D), lambda i: (0, 0)),
        ],
        out_specs=[
            pl.BlockSpec((RB, D), lambda i: (i, 0)),
            pl.BlockSpec((RB, D), lambda i: (i, 0)),
        ],
        out_shape=[jax.ShapeDtypeStruct((N, D), jnp.float32)] * 2,
    )(deg, agg1, z1, b1, g1, be1, W2)


def _ep2_body(deg_ref, agg_ref, z_ref, b_ref, g_ref, be_ref,
              h1_ref, out_ref):
    dv = lax.rsqrt(1.0 + deg_ref[...])
    pre = dv * (agg_ref[...] + z_ref[...]) + b_ref[...]
    h2 = _ln_relu(pre, g_ref, be_ref)
    out_ref[...] = h1_ref[...] + h2


def _ep2(deg, agg2, z2, b2, g2, be2, h1):
    return pl.pallas_call(
        _ep2_body,
        grid=(N // RB,),
        in_specs=[
            pl.BlockSpec((RB, 1), lambda i: (i, 0)),
            pl.BlockSpec((RB, D), lambda i: (i, 0)),
            pl.BlockSpec((RB, D), lambda i: (i, 0)),
            pl.BlockSpec((1, D), lambda i: (0, 0)),
            pl.BlockSpec((1, D), lambda i: (0, 0)),
            pl.BlockSpec((1, D), lambda i: (0, 0)),
            pl.BlockSpec((RB, D), lambda i: (i, 0)),
        ],
        out_specs=pl.BlockSpec((RB, D), lambda i: (i, 0)),
        out_shape=jax.ShapeDtypeStruct((N, D), jnp.float32),
    )(deg, agg2, z2, b2, g2, be2, h1)


def kernel(x, edge_index, W1, b1, g1, be1, W2, b2, g2, be2):
    src_flat = edge_index[0]
    dst_flat = edge_index[1]

    csrc, cdst, cnts, deg = _compact_kernel(src_flat, dst_flat)
    deg2 = deg.reshape(N, 1)

    z1 = _mm(deg2, x, W1)
    agg1 = _agg_kernel(z1, csrc, cdst, cnts).reshape(N, D)
    h1, z2 = _ep1(deg2, agg1, z1, b1[None], g1[None], be1[None], W2)
    agg2 = _agg_kernel(z2, csrc, cdst, cnts).reshape(N, D)
    return _ep2(deg2, agg2, z2, b2[None], g2[None], be2[None], h1)


# speedup vs baseline: 6.6771x; 1.1155x over previous
import functools

import jax
import jax.numpy as jnp
from jax import lax
from jax.experimental import pallas as pl
from jax.experimental.pallas import tpu as pltpu
from jax.experimental.pallas import tpu_sc as plsc

N = 10000
D = 256
E = 160000

NC = 2
NS = 16
NW = NC * NS
RNG = 312
RNGL = N - (NW - 1) * RNG
ACCR = 336
TS = ACCR - 1
SCH = 2000
NSCH = E // SCH
STG = 4096
FLUSH = 2048
EPW = E + FLUSH
GCH = 64

RB = 1000

_mesh = plsc.VectorSubcoreMesh(
    core_axis_name="c", subcore_axis_name="s", num_cores=NC, num_subcores=NS
)
_params = pltpu.CompilerParams(needs_layout_passes=False)


def _worker(c, t):
    return c * NS + t


@functools.partial(
    pl.kernel,
    out_type=[
        jax.ShapeDtypeStruct((NW * EPW,), jnp.int32),
        jax.ShapeDtypeStruct((NW * EPW,), jnp.int32),
        jax.ShapeDtypeStruct((NW * 16,), jnp.int32),
        jax.ShapeDtypeStruct((N,), jnp.float32),
    ],
    mesh=_mesh,
    compiler_params=_params,
    scratch_types=[
        pltpu.VMEM((SCH,), jnp.int32),
        pltpu.VMEM((SCH,), jnp.int32),
        pltpu.VMEM((SCH,), jnp.int32),
        pltpu.VMEM((SCH,), jnp.int32),
        pltpu.VMEM((STG,), jnp.int32),
        pltpu.VMEM((STG,), jnp.int32),
        pltpu.VMEM((ACCR,), jnp.float32),
        pltpu.SemaphoreType.DMA,
        pltpu.SemaphoreType.DMA,
    ],
)
def _compact_kernel(src_hbm, dst_hbm, csrc_hbm, cdst_hbm, cnts_hbm, deg_hbm,
                    sbuf0, sbuf1, dbuf0, dbuf1, stg_s, stg_d, degloc,
                    sem_c0, sem_c1):
    c = lax.axis_index("c")
    t = lax.axis_index("s")
    w = _worker(c, t)
    lo = w * RNG
    hi = jnp.where(w == NW - 1, N, lo + RNG)
    base = w * EPW
    ones16 = jnp.ones((16,), jnp.float32)
    zero16 = jnp.zeros((16,), jnp.float32)

    def zdeg(i, _):
        degloc[pl.ds(i * 16, 16)] = zero16
        return 0

    lax.fori_loop(0, ACCR // 16, zdeg, 0)

    def fetch_chunk(k, sb, db, sem):
        pltpu.async_copy(src_hbm.at[pl.ds(k * SCH, SCH)], sb, sem)
        pltpu.async_copy(dst_hbm.at[pl.ds(k * SCH, SCH)], db, sem)

    def wait_chunk(sb, db, sem):
        pltpu.make_async_copy(src_hbm.at[pl.ds(0, SCH)], sb, sem).wait()
        pltpu.make_async_copy(dst_hbm.at[pl.ds(0, SCH)], db, sem).wait()

    fetch_chunk(0, sbuf0, dbuf0, sem_c0)
    fetch_chunk(1, sbuf1, dbuf1, sem_c1)

    def chunk(k, sbuf, dbuf, sem, carry):
        off0, hoff0 = carry
        wait_chunk(sbuf, dbuf, sem)

        def grp(j, off):
            d16 = dbuf[pl.ds(j * 16, 16)]
            s16 = sbuf[pl.ds(j * 16, 16)]
            dl = d16 - lo
            m = (d16 >= lo) & (d16 < hi)
            plsc.store_compressed(stg_s.at[pl.ds(off, 16)], s16, mask=m)
            plsc.store_compressed(stg_d.at[pl.ds(off, 16)], dl, mask=m)
            plsc.addupdate_scatter(
                degloc, [jnp.where(m, dl, TS)], ones16, mask=m)
            pc = plsc.all_reduce_population_count(m)
            return off + pc[0]

        off = lax.fori_loop(0, SCH // 16, grp, off0)

        @pl.when(k + 2 < NSCH)
        def _():
            fetch_chunk(k + 2, sbuf, dbuf, sem)

        def do_flush(a):
            o, h = a
            ho = pl.multiple_of(base + h, 8)
            pltpu.sync_copy(stg_s.at[pl.ds(0, FLUSH)],
                            csrc_hbm.at[pl.ds(ho, FLUSH)])
            pltpu.sync_copy(stg_d.at[pl.ds(0, FLUSH)],
                            cdst_hbm.at[pl.ds(ho, FLUSH)])

            def shift(i, _):
                stg_s[pl.ds(i * 16, 16)] = stg_s[pl.ds(FLUSH + i * 16, 16)]
                stg_d[pl.ds(i * 16, 16)] = stg_d[pl.ds(FLUSH + i * 16, 16)]
                return 0

            lax.fori_loop(0, FLUSH // 16, shift, 0)
            return (o - FLUSH, h + FLUSH)

        return lax.cond(off >= FLUSH, do_flush, lambda a: a, (off, hoff0))

    def chunk_pair(p, carry):
        carry = chunk(2 * p, sbuf0, dbuf0, sem_c0, carry)
        return chunk(2 * p + 1, sbuf1, dbuf1, sem_c1, carry)

    off, hoff = lax.fori_loop(0, NSCH // 2, chunk_pair,
                              (jnp.int32(0), jnp.int32(0)))

    hof = pl.multiple_of(base + hoff, 8)
    pltpu.sync_copy(stg_s.at[pl.ds(0, FLUSH)],
                    csrc_hbm.at[pl.ds(hof, FLUSH)])
    pltpu.sync_copy(stg_d.at[pl.ds(0, FLUSH)],
                    cdst_hbm.at[pl.ds(hof, FLUSH)])

    sbuf0[pl.ds(0, 16)] = jnp.broadcast_to(hoff + off, (16,))
    pltpu.sync_copy(sbuf0.at[pl.ds(0, 16)], cnts_hbm.at[pl.ds(w * 16, 16)])

    @pl.when(w < NW - 1)
    def _():
        pltpu.sync_copy(degloc.at[pl.ds(0, RNG)],
                        deg_hbm.at[pl.ds(lo, RNG)])

    @pl.when(w == NW - 1)
    def _():
        pltpu.sync_copy(degloc.at[pl.ds(0, RNGL)],
                        deg_hbm.at[pl.ds(lo, RNGL)])


@functools.partial(
    pl.kernel,
    out_type=jax.ShapeDtypeStruct((N * D,), jnp.float32),
    mesh=_mesh,
    compiler_params=_params,
    scratch_types=[
        pltpu.VMEM((160,), jnp.int32),
        pltpu.VMEM((272,), jnp.int32),
        pltpu.VMEM((2, GCH, D), jnp.float32),
        pltpu.VMEM((ACCR * D,), jnp.float32),
        pltpu.SemaphoreType.DMA,
        pltpu.SemaphoreType.DMA,
        pltpu.SemaphoreType.DMA,
    ],
)
def _agg_kernel(z_hbm, csrc_hbm, cdst_hbm, cnts_hbm, agg_hbm,
                srcb, dstb, rows, acc, sem_a, sem_b, sem_i):
    c = lax.axis_index("c")
    t = lax.axis_index("s")
    w = _worker(c, t)
    base = w * EPW
    iota16 = lax.iota(jnp.int32, 16)
    zero16 = jnp.zeros((16,), jnp.float32)
    lanes = D // 16

    pltpu.sync_copy(cnts_hbm.at[pl.ds(w * 16, 16)], srcb.at[pl.ds(0, 16)])
    cnt = srcb[pl.ds(0, 16)][0]
    nch = (cnt + (GCH - 1)) // GCH

    def zacc(i, _):
        acc[pl.ds(i * 16, 16)] = zero16
        return 0

    lax.fori_loop(0, ACCR * lanes, zacc, 0)

    def fetch_idx(g, sb, db, sync):
        go = pl.multiple_of(base + g * GCH, 8)
        if sync:
            pltpu.sync_copy(csrc_hbm.at[pl.ds(go, GCH)],
                            srcb.at[pl.ds(GCH * sb, GCH)])
            pltpu.sync_copy(cdst_hbm.at[pl.ds(go, GCH)],
                            dstb.at[pl.ds(GCH * db, GCH)])
        else:
            pltpu.async_copy(csrc_hbm.at[pl.ds(go, GCH)],
                             srcb.at[pl.ds(GCH * sb, GCH)], sem_i)
            pltpu.async_copy(cdst_hbm.at[pl.ds(go, GCH)],
                             dstb.at[pl.ds(GCH * db, GCH)], sem_i)

    def wait_idx(sb, db):
        pltpu.make_async_copy(csrc_hbm.at[pl.ds(base, GCH)],
                              srcb.at[pl.ds(GCH * sb, GCH)], sem_i).wait()
        pltpu.make_async_copy(cdst_hbm.at[pl.ds(base, GCH)],
                              dstb.at[pl.ds(GCH * db, GCH)], sem_i).wait()

    def remap(g, sb, db):
        for q in range(GCH // 16):
            ssl = pl.ds(GCH * sb + q * 16, 16)
            dsl = pl.ds(GCH * db + q * 16, 16)
            pos = g * GCH + q * 16 + iota16
            mm = pos < cnt
            srcb[ssl] = jnp.where(mm, srcb[ssl], iota16 & 7)
            dstb[dsl] = jnp.where(mm, dstb[dsl], TS)

    def start_gather(sb, sem):
        pltpu.async_copy(z_hbm.at[srcb.at[pl.ds(GCH * sb, GCH)]],
                         rows.at[sb], sem)

    def wait_gather(sb, sem):
        pltpu.make_async_copy(z_hbm.at[srcb.at[pl.ds(GCH * sb, GCH)]],
                              rows.at[sb], sem).wait()

    def acc_chunk(sb, db):
        def acc_q(q, _):
            slots16 = dstb[pl.ds(GCH * db + q * 16, 16)]
            for e in range(16):
                sl = slots16[e]
                be = sl * D + iota16
                for j in range(lanes):
                    plsc.addupdate_scatter(
                        acc, [be + (j * 16)],
                        rows[sb, q * 16 + e, pl.ds(j * 16, 16)])
            return 0

        lax.fori_loop(0, GCH // 16, acc_q, 0)

    @pl.when(nch > 0)
    def _():
        fetch_idx(0, 0, 0, True)
        remap(0, 0, 0)
        start_gather(0, sem_a)

    @pl.when(nch > 1)
    def _():
        fetch_idx(1, 1, 1, True)
        remap(1, 1, 1)
        start_gather(1, sem_b)

    def quad(p4, _):
        for k in range(4):
            g = 4 * p4 + k
            sb = k % 2
            db = k
            nd = (k + 2) % 4
            sem = sem_a if sb == 0 else sem_b

            @pl.when(g < nch)
            def _():
                wait_gather(sb, sem)

                @pl.when(g + 2 < nch)
                def _():
                    fetch_idx(g + 2, sb, nd, False)

                acc_chunk(sb, db)

                @pl.when(g + 2 < nch)
                def _():
                    wait_idx(sb, nd)
                    remap(g + 2, sb, nd)
                    start_gather(sb, sem)

        return 0

    lax.fori_loop(0, (nch + 3) // 4, quad, 0)

    @pl.when(w < NW - 1)
    def _():
        pltpu.sync_copy(acc.at[pl.ds(0, RNG * D)],
                        agg_hbm.at[pl.ds(w * (RNG * D), RNG * D)])

    @pl.when(w == NW - 1)
    def _():
        pltpu.sync_copy(acc.at[pl.ds(0, RNGL * D)],
                        agg_hbm.at[pl.ds(w * (RNG * D), RNGL * D)])


def _mm_body(deg_ref, x_ref, w_ref, z_ref):
    dv = lax.rsqrt(1.0 + deg_ref[...])
    z_ref[...] = jnp.dot(
        x_ref[...] * dv, w_ref[...], preferred_element_type=jnp.float32
    )


def _mm(deg, x, w):
    return pl.pallas_call(
        _mm_body,
        grid=(N // RB,),
        in_specs=[
            pl.BlockSpec((RB, 1), lambda i: (i, 0)),
            pl.BlockSpec((RB, D), lambda i: (i, 0)),
            pl.BlockSpec((D, D), lambda i: (0, 0)),
        ],
        out_specs=pl.BlockSpec((RB, D), lambda i: (i, 0)),
        out_shape=jax.ShapeDtypeStruct((N, D), jnp.float32),
    )(deg, x, w)


def _ln_relu(pre, g_ref, be_ref):
    mu = jnp.mean(pre, axis=-1, keepdims=True)
    xc = pre - mu
    var = jnp.mean(xc * xc, axis=-1, keepdims=True)
    y = xc * lax.rsqrt(var + 1e-5) * g_ref[...] + be_ref[...]
    return jnp.maximum(y, 0.0)


def _ep1_body(deg_ref, agg_ref, z_ref, b_ref, g_ref, be_ref,
              w2_ref, h1_ref, z2_ref):
    dv = lax.rsqrt(1.0 + deg_ref[...])
    pre = dv * (agg_ref[...] + z_ref[...]) + b_ref[...]
    h1 = _ln_relu(pre, g_ref, be_ref)
    h1_ref[...] = h1
    z2_ref[...] = jnp.dot(
        h1 * dv, w2_ref[...], preferred_element_type=jnp.float32
    )


def _ep1(deg, agg1, z1, b1, g1, be1, W2):
    return pl.pallas_call(
        _ep1_body,
        grid=(N // RB,),
        in_specs=[
            pl.BlockSpec((RB, 1), lambda i: (i, 0)),
            pl.BlockSpec((RB, D), lambda i: (i, 0)),
            pl.BlockSpec((RB, D), lambda i: (i, 0)),
            pl.BlockSpec((1, D), lambda i: (0, 0)),
            pl.BlockSpec((1, D), lambda i: (0, 0)),
            pl.BlockSpec((1, D), lambda i: (0, 0)),
            pl.BlockSpec((D, D), lambda i: (0, 0)),
        ],
        out_specs=[
            pl.BlockSpec((RB, D), lambda i: (i, 0)),
            pl.BlockSpec((RB, D), lambda i: (i, 0)),
        ],
        out_shape=[jax.ShapeDtypeStruct((N, D), jnp.float32)] * 2,
    )(deg, agg1, z1, b1, g1, be1, W2)


def _ep2_body(deg_ref, agg_ref, z_ref, b_ref, g_ref, be_ref,
              h1_ref, out_ref):
    dv = lax.rsqrt(1.0 + deg_ref[...])
    pre = dv * (agg_ref[...] + z_ref[...]) + b_ref[...]
    h2 = _ln_relu(pre, g_ref, be_ref)
    out_ref[...] = h1_ref[...] + h2


def _ep2(deg, agg2, z2, b2, g2, be2, h1):
    return pl.pallas_call(
        _ep2_body,
        grid=(N // RB,),
        in_specs=[
            pl.BlockSpec((RB, 1), lambda i: (i, 0)),
            pl.BlockSpec((RB, D), lambda i: (i, 0)),
            pl.BlockSpec((RB, D), lambda i: (i, 0)),
            pl.BlockSpec((1, D), lambda i: (0, 0)),
            pl.BlockSpec((1, D), lambda i: (0, 0)),
            pl.BlockSpec((1, D), lambda i: (0, 0)),
            pl.BlockSpec((RB, D), lambda i: (i, 0)),
        ],
        out_specs=pl.BlockSpec((RB, D), lambda i: (i, 0)),
        out_shape=jax.ShapeDtypeStruct((N, D), jnp.float32),
    )(deg, agg2, z2, b2, g2, be2, h1)


def kernel(x, edge_index, W1, b1, g1, be1, W2, b2, g2, be2):
    src_flat = edge_index[0]
    dst_flat = edge_index[1]

    csrc, cdst, cnts, deg = _compact_kernel(src_flat, dst_flat)
    deg2 = deg.reshape(N, 1)

    z1 = _mm(deg2, x, W1)
    agg1 = _agg_kernel(z1, csrc, cdst, cnts).reshape(N, D)
    h1, z2 = _ep1(deg2, agg1, z1, b1[None], g1[None], be1[None], W2)
    agg2 = _agg_kernel(z2, csrc, cdst, cnts).reshape(N, D)
    return _ep2(deg2, agg2, z2, b2[None], g2[None], be2[None], h1)


# speedup vs baseline: 6.7077x; 1.0046x over previous
import functools

import jax
import jax.numpy as jnp
from jax import lax
from jax.experimental import pallas as pl
from jax.experimental.pallas import tpu as pltpu
from jax.experimental.pallas import tpu_sc as plsc

N = 10000
D = 256
E = 160000

NC = 2
NS = 16
NW = NC * NS
RNG = 312
RNGL = N - (NW - 1) * RNG
ACCR = 336
TS = ACCR - 1
SCH = 2000
NSCH = E // SCH
STG = 4096
FLUSH = 2048
EPW = E + FLUSH
GCH = 80

RB = 1000

_mesh = plsc.VectorSubcoreMesh(
    core_axis_name="c", subcore_axis_name="s", num_cores=NC, num_subcores=NS
)
_params = pltpu.CompilerParams(needs_layout_passes=False)


def _worker(c, t):
    return c * NS + t


@functools.partial(
    pl.kernel,
    out_type=[
        jax.ShapeDtypeStruct((NW * EPW,), jnp.int32),
        jax.ShapeDtypeStruct((NW * EPW,), jnp.int32),
        jax.ShapeDtypeStruct((NW * 16,), jnp.int32),
        jax.ShapeDtypeStruct((N,), jnp.float32),
    ],
    mesh=_mesh,
    compiler_params=_params,
    scratch_types=[
        pltpu.VMEM((SCH,), jnp.int32),
        pltpu.VMEM((SCH,), jnp.int32),
        pltpu.VMEM((SCH,), jnp.int32),
        pltpu.VMEM((SCH,), jnp.int32),
        pltpu.VMEM((STG,), jnp.int32),
        pltpu.VMEM((STG,), jnp.int32),
        pltpu.VMEM((ACCR,), jnp.float32),
        pltpu.SemaphoreType.DMA,
        pltpu.SemaphoreType.DMA,
    ],
)
def _compact_kernel(src_hbm, dst_hbm, csrc_hbm, cdst_hbm, cnts_hbm, deg_hbm,
                    sbuf0, sbuf1, dbuf0, dbuf1, stg_s, stg_d, degloc,
                    sem_c0, sem_c1):
    c = lax.axis_index("c")
    t = lax.axis_index("s")
    w = _worker(c, t)
    lo = w * RNG
    hi = jnp.where(w == NW - 1, N, lo + RNG)
    base = w * EPW
    ones16 = jnp.ones((16,), jnp.float32)
    zero16 = jnp.zeros((16,), jnp.float32)

    def zdeg(i, _):
        degloc[pl.ds(i * 16, 16)] = zero16
        return 0

    lax.fori_loop(0, ACCR // 16, zdeg, 0)

    def fetch_chunk(k, sb, db, sem):
        pltpu.async_copy(src_hbm.at[pl.ds(k * SCH, SCH)], sb, sem)
        pltpu.async_copy(dst_hbm.at[pl.ds(k * SCH, SCH)], db, sem)

    def wait_chunk(sb, db, sem):
        pltpu.make_async_copy(src_hbm.at[pl.ds(0, SCH)], sb, sem).wait()
        pltpu.make_async_copy(dst_hbm.at[pl.ds(0, SCH)], db, sem).wait()

    fetch_chunk(0, sbuf0, dbuf0, sem_c0)
    fetch_chunk(1, sbuf1, dbuf1, sem_c1)

    def chunk(k, sbuf, dbuf, sem, carry):
        off0, hoff0 = carry
        wait_chunk(sbuf, dbuf, sem)

        def grp(j, off):
            d16 = dbuf[pl.ds(j * 16, 16)]
            s16 = sbuf[pl.ds(j * 16, 16)]
            dl = d16 - lo
            m = (d16 >= lo) & (d16 < hi)
            plsc.store_compressed(stg_s.at[pl.ds(off, 16)], s16, mask=m)
            plsc.store_compressed(stg_d.at[pl.ds(off, 16)], dl, mask=m)
            plsc.addupdate_scatter(
                degloc, [jnp.where(m, dl, TS)], ones16, mask=m)
            pc = plsc.all_reduce_population_count(m)
            return off + pc[0]

        off = lax.fori_loop(0, SCH // 16, grp, off0)

        @pl.when(k + 2 < NSCH)
        def _():
            fetch_chunk(k + 2, sbuf, dbuf, sem)

        def do_flush(a):
            o, h = a
            ho = pl.multiple_of(base + h, 8)
            pltpu.sync_copy(stg_s.at[pl.ds(0, FLUSH)],
                            csrc_hbm.at[pl.ds(ho, FLUSH)])
            pltpu.sync_copy(stg_d.at[pl.ds(0, FLUSH)],
                            cdst_hbm.at[pl.ds(ho, FLUSH)])

            def shift(i, _):
                stg_s[pl.ds(i * 16, 16)] = stg_s[pl.ds(FLUSH + i * 16, 16)]
                stg_d[pl.ds(i * 16, 16)] = stg_d[pl.ds(FLUSH + i * 16, 16)]
                return 0

            lax.fori_loop(0, FLUSH // 16, shift, 0)
            return (o - FLUSH, h + FLUSH)

        return lax.cond(off >= FLUSH, do_flush, lambda a: a, (off, hoff0))

    def chunk_pair(p, carry):
        carry = chunk(2 * p, sbuf0, dbuf0, sem_c0, carry)
        return chunk(2 * p + 1, sbuf1, dbuf1, sem_c1, carry)

    off, hoff = lax.fori_loop(0, NSCH // 2, chunk_pair,
                              (jnp.int32(0), jnp.int32(0)))

    hof = pl.multiple_of(base + hoff, 8)
    pltpu.sync_copy(stg_s.at[pl.ds(0, FLUSH)],
                    csrc_hbm.at[pl.ds(hof, FLUSH)])
    pltpu.sync_copy(stg_d.at[pl.ds(0, FLUSH)],
                    cdst_hbm.at[pl.ds(hof, FLUSH)])

    sbuf0[pl.ds(0, 16)] = jnp.broadcast_to(hoff + off, (16,))
    pltpu.sync_copy(sbuf0.at[pl.ds(0, 16)], cnts_hbm.at[pl.ds(w * 16, 16)])

    @pl.when(w < NW - 1)
    def _():
        pltpu.sync_copy(degloc.at[pl.ds(0, RNG)],
                        deg_hbm.at[pl.ds(lo, RNG)])

    @pl.when(w == NW - 1)
    def _():
        pltpu.sync_copy(degloc.at[pl.ds(0, RNGL)],
                        deg_hbm.at[pl.ds(lo, RNGL)])


@functools.partial(
    pl.kernel,
    out_type=jax.ShapeDtypeStruct((N * D,), jnp.float32),
    mesh=_mesh,
    compiler_params=_params,
    scratch_types=[
        pltpu.VMEM((160,), jnp.int32),
        pltpu.VMEM((4 * GCH + 16,), jnp.int32),
        pltpu.VMEM((2, GCH, D), jnp.float32),
        pltpu.VMEM((ACCR * D,), jnp.float32),
        pltpu.SemaphoreType.DMA,
        pltpu.SemaphoreType.DMA,
        pltpu.SemaphoreType.DMA,
    ],
)
def _agg_kernel(z_hbm, csrc_hbm, cdst_hbm, cnts_hbm, agg_hbm,
                srcb, dstb, rows, acc, sem_a, sem_b, sem_i):
    c = lax.axis_index("c")
    t = lax.axis_index("s")
    w = _worker(c, t)
    base = w * EPW
    iota16 = lax.iota(jnp.int32, 16)
    zero16 = jnp.zeros((16,), jnp.float32)
    lanes = D // 16

    pltpu.sync_copy(cnts_hbm.at[pl.ds(w * 16, 16)], srcb.at[pl.ds(0, 16)])
    cnt = srcb[pl.ds(0, 16)][0]
    nch = (cnt + (GCH - 1)) // GCH

    def zacc(i, _):
        acc[pl.ds(i * 16, 16)] = zero16
        return 0

    lax.fori_loop(0, ACCR * lanes, zacc, 0)

    def fetch_idx(g, sb, db, sync):
        go = pl.multiple_of(base + g * GCH, 8)
        if sync:
            pltpu.sync_copy(csrc_hbm.at[pl.ds(go, GCH)],
                            srcb.at[pl.ds(GCH * sb, GCH)])
            pltpu.sync_copy(cdst_hbm.at[pl.ds(go, GCH)],
                            dstb.at[pl.ds(GCH * db, GCH)])
        else:
            pltpu.async_copy(csrc_hbm.at[pl.ds(go, GCH)],
                             srcb.at[pl.ds(GCH * sb, GCH)], sem_i)
            pltpu.async_copy(cdst_hbm.at[pl.ds(go, GCH)],
                             dstb.at[pl.ds(GCH * db, GCH)], sem_i)

    def wait_idx(sb, db):
        pltpu.make_async_copy(csrc_hbm.at[pl.ds(base, GCH)],
                              srcb.at[pl.ds(GCH * sb, GCH)], sem_i).wait()
        pltpu.make_async_copy(cdst_hbm.at[pl.ds(base, GCH)],
                              dstb.at[pl.ds(GCH * db, GCH)], sem_i).wait()

    def remap(g, sb, db):
        for q in range(GCH // 16):
            ssl = pl.ds(GCH * sb + q * 16, 16)
            dsl = pl.ds(GCH * db + q * 16, 16)
            pos = g * GCH + q * 16 + iota16
            mm = pos < cnt
            srcb[ssl] = jnp.where(mm, srcb[ssl], iota16 & 7)
            dstb[dsl] = jnp.where(mm, dstb[dsl], TS)

    def start_gather(sb, sem):
        pltpu.async_copy(z_hbm.at[srcb.at[pl.ds(GCH * sb, GCH)]],
                         rows.at[sb], sem)

    def wait_gather(sb, sem):
        pltpu.make_async_copy(z_hbm.at[srcb.at[pl.ds(GCH * sb, GCH)]],
                              rows.at[sb], sem).wait()

    def acc_chunk(sb, db):
        def acc_q(q, _):
            slots16 = dstb[pl.ds(GCH * db + q * 16, 16)]
            for e in range(16):
                sl = slots16[e]
                be = sl * D + iota16
                for j in range(lanes):
                    plsc.addupdate_scatter(
                        acc, [be + (j * 16)],
                        rows[sb, q * 16 + e, pl.ds(j * 16, 16)])
            return 0

        lax.fori_loop(0, GCH // 16, acc_q, 0)

    @pl.when(nch > 0)
    def _():
        fetch_idx(0, 0, 0, True)
        remap(0, 0, 0)
        start_gather(0, sem_a)

    @pl.when(nch > 1)
    def _():
        fetch_idx(1, 1, 1, True)
        remap(1, 1, 1)
        start_gather(1, sem_b)

    def quad(p4, _):
        for k in range(4):
            g = 4 * p4 + k
            sb = k % 2
            db = k
            nd = (k + 2) % 4
            sem = sem_a if sb == 0 else sem_b

            @pl.when(g < nch)
            def _():
                wait_gather(sb, sem)

                @pl.when(g + 2 < nch)
                def _():
                    fetch_idx(g + 2, sb, nd, False)

                acc_chunk(sb, db)

                @pl.when(g + 2 < nch)
                def _():
                    wait_idx(sb, nd)
                    remap(g + 2, sb, nd)
                    start_gather(sb, sem)

        return 0

    lax.fori_loop(0, (nch + 3) // 4, quad, 0)

    @pl.when(w < NW - 1)
    def _():
        pltpu.sync_copy(acc.at[pl.ds(0, RNG * D)],
                        agg_hbm.at[pl.ds(w * (RNG * D), RNG * D)])

    @pl.when(w == NW - 1)
    def _():
        pltpu.sync_copy(acc.at[pl.ds(0, RNGL * D)],
                        agg_hbm.at[pl.ds(w * (RNG * D), RNGL * D)])


def _mm_body(deg_ref, x_ref, w_ref, z_ref):
    dv = lax.rsqrt(1.0 + deg_ref[...])
    z_ref[...] = jnp.dot(
        x_ref[...] * dv, w_ref[...], preferred_element_type=jnp.float32
    )


def _mm(deg, x, w):
    return pl.pallas_call(
        _mm_body,
        grid=(N // RB,),
        in_specs=[
            pl.BlockSpec((RB, 1), lambda i: (i, 0)),
            pl.BlockSpec((RB, D), lambda i: (i, 0)),
            pl.BlockSpec((D, D), lambda i: (0, 0)),
        ],
        out_specs=pl.BlockSpec((RB, D), lambda i: (i, 0)),
        out_shape=jax.ShapeDtypeStruct((N, D), jnp.float32),
    )(deg, x, w)


def _ln_relu(pre, g_ref, be_ref):
    mu = jnp.mean(pre, axis=-1, keepdims=True)
    xc = pre - mu
    var = jnp.mean(xc * xc, axis=-1, keepdims=True)
    y = xc * lax.rsqrt(var + 1e-5) * g_ref[...] + be_ref[...]
    return jnp.maximum(y, 0.0)


def _ep1_body(deg_ref, agg_ref, z_ref, b_ref, g_ref, be_ref,
              w2_ref, h1_ref, z2_ref):
    dv = lax.rsqrt(1.0 + deg_ref[...])
    pre = dv * (agg_ref[...] + z_ref[...]) + b_ref[...]
    h1 = _ln_relu(pre, g_ref, be_ref)
    h1_ref[...] = h1
    z2_ref[...] = jnp.dot(
        h1 * dv, w2_ref[...], preferred_element_type=jnp.float32
    )


def _ep1(deg, agg1, z1, b1, g1, be1, W2):
    return pl.pallas_call(
        _ep1_body,
        grid=(N // RB,),
        in_specs=[
            pl.BlockSpec((RB, 1), lambda i: (i, 0)),
            pl.BlockSpec((RB, D), lambda i: (i, 0)),
            pl.BlockSpec((RB, D), lambda i: (i, 0)),
            pl.BlockSpec((1, D), lambda i: (0, 0)),
            pl.BlockSpec((1, D), lambda i: (0, 0)),
            pl.BlockSpec((1, D), lambda i: (0, 0)),
            pl.BlockSpec((D, D), lambda i: (0, 0)),
        ],
        out_specs=[
            pl.BlockSpec((RB, D), lambda i: (i, 0)),
            pl.BlockSpec((RB, D), lambda i: (i, 0)),
        ],
        out_shape=[jax.ShapeDtypeStruct((N, D), jnp.float32)] * 2,
    )(deg, agg1, z1, b1, g1, be1, W2)


def _ep2_body(deg_ref, agg_ref, z_ref, b_ref, g_ref, be_ref,
              h1_ref, out_ref):
    dv = lax.rsqrt(1.0 + deg_ref[...])
    pre = dv * (agg_ref[...] + z_ref[...]) + b_ref[...]
    h2 = _ln_relu(pre, g_ref, be_ref)
    out_ref[...] = h1_ref[...] + h2


def _ep2(deg, agg2, z2, b2, g2, be2, h1):
    return pl.pallas_call(
        _ep2_body,
        grid=(N // RB,),
        in_specs=[
            pl.BlockSpec((RB, 1), lambda i: (i, 0)),
            pl.BlockSpec((RB, D), lambda i: (i, 0)),
            pl.BlockSpec((RB, D), lambda i: (i, 0)),
            pl.BlockSpec((1, D), lambda i: (0, 0)),
            pl.BlockSpec((1, D), lambda i: (0, 0)),
            pl.BlockSpec((1, D), lambda i: (0, 0)),
            pl.BlockSpec((RB, D), lambda i: (i, 0)),
        ],
        out_specs=pl.BlockSpec((RB, D), lambda i: (i, 0)),
        out_shape=jax.ShapeDtypeStruct((N, D), jnp.float32),
    )(deg, agg2, z2, b2, g2, be2, h1)


def kernel(x, edge_index, W1, b1, g1, be1, W2, b2, g2, be2):
    src_flat = edge_index[0]
    dst_flat = edge_index[1]

    csrc, cdst, cnts, deg = _compact_kernel(src_flat, dst_flat)
    deg2 = deg.reshape(N, 1)

    z1 = _mm(deg2, x, W1)
    agg1 = _agg_kernel(z1, csrc, cdst, cnts).reshape(N, D)
    h1, z2 = _ep1(deg2, agg1, z1, b1[None], g1[None], be1[None], W2)
    agg2 = _agg_kernel(z2, csrc, cdst, cnts).reshape(N, D)
    return _ep2(deg2, agg2, z2, b2[None], g2[None], be2[None], h1)
